# Initial kernel scaffold; baseline (speedup 1.0000x reference)
import jax
import jax.numpy as jnp
from jax.experimental import pallas as pl


def kernel(x, edge_index, v_edge_index, batch, W_rel_d1, b_rel_d1, W_root_d1, W_rel_u1, b_rel_u1, W_root_u1, W_rel_d2, b_rel_d2, W_root_d2, W_rel_u2, b_rel_u2, W_root_u2, W_cls, b_cls):
    raise NotImplementedError("write your pallas kernel here")


# speedup vs baseline: 58.7486x; 58.7486x over previous
import functools

import jax
import jax.numpy as jnp
from jax import lax
from jax.experimental import pallas as pl
from jax.experimental.pallas import tpu as pltpu
from jax.experimental.pallas import tpu_sc as plsc

N = 50000
E = 3200000
EV = 1600000
G = 128
H = 16
C = 2

NCORES = 2
NTILES = 16
NPT = 3200
N_PAD = NTILES * NPT
CH = 2048
SUB = 128
NSUB = CH // SUB
ZR = 400

E_PAD = 32 * 49 * CH
EV_PAD = 32 * 25 * CH
EPT_A = E_PAD // 32
EVPT_A = EV_PAD // 32
EPT_B = E_PAD // 16
EVPT_B = EV_PAD // 16

def _scalar_segsum_body(x_hbm, srcE, dstE, srcV, dstV, se_out, sv_out,
                        sidx, didx, vals, zbuf, x_sp, acc_e, acc_v, sem):
    cid = lax.axis_index("c")
    sid = lax.axis_index("s")

    def _zero(i, _):
        zbuf[pl.ds(pl.multiple_of(i * 16, 16), 16)] = jnp.zeros((16,), jnp.float32)
        return 0
    lax.fori_loop(0, NPT // 16, _zero, 0)
    sl = pl.ds(pl.multiple_of(sid * NPT, NPT), NPT)
    pltpu.sync_copy(zbuf, acc_e.at[sl])
    pltpu.sync_copy(zbuf, acc_v.at[sl])
    pltpu.sync_copy(x_hbm.at[sl], zbuf)
    pltpu.sync_copy(zbuf, x_sp.at[sl])
    plsc.subcore_barrier()

    tile = cid * NTILES + sid

    def _edge_chunks(src_hbm, dst_hbm, acc, edge_off, nchunks):
        def body(i, _):
            row0 = pl.multiple_of((edge_off + i * CH) // SUB, NSUB)
            pltpu.sync_copy(src_hbm.at[pl.ds(row0, NSUB)], sidx)
            pltpu.sync_copy(dst_hbm.at[pl.ds(row0, NSUB)], didx)
            cps = [pltpu.async_copy(x_sp.at[sidx.at[j]], vals.at[j], sem)
                   for j in range(NSUB)]
            for cp in cps:
                cp.wait()
            for j in range(NSUB):
                pltpu.sync_copy(vals.at[j], acc.at[didx.at[j]], add=True)
            return 0
        lax.fori_loop(0, nchunks, body, 0)

    _edge_chunks(srcE, dstE, acc_e, tile * EPT_A, EPT_A // CH)
    _edge_chunks(srcV, dstV, acc_v, tile * EVPT_A, EVPT_A // CH)
    plsc.subcore_barrier()

    pltpu.sync_copy(acc_e.at[sl], zbuf)
    pltpu.sync_copy(zbuf, se_out.at[cid].at[sl])
    pltpu.sync_copy(acc_v.at[sl], zbuf)
    pltpu.sync_copy(zbuf, sv_out.at[cid].at[sl])


def _row_segsum_body(h_hbm, srcE, dstE, srcV, dstV, aggE_out, aggV_out,
                     sidx, didx, rows, zrow, acc, sem):
    cid = lax.axis_index("c")
    sid = lax.axis_index("s")

    def _zero(i, _):
        zrow[i, :] = jnp.zeros((16,), jnp.float32)
        return 0
    lax.fori_loop(0, ZR, _zero, 0)
    for k in range(NPT // ZR):
        pltpu.sync_copy(
            zrow, acc.at[pl.ds(pl.multiple_of(sid * NPT + k * ZR, ZR), ZR)])
    plsc.subcore_barrier()

    def _edge_chunks(src_hbm, dst_hbm, edge_off, nchunks):
        def body(i, _):
            row0 = pl.multiple_of((edge_off + i * CH) // SUB, NSUB)
            pltpu.sync_copy(src_hbm.at[pl.ds(row0, NSUB)], sidx)
            pltpu.sync_copy(dst_hbm.at[pl.ds(row0, NSUB)], didx)
            cps = [pltpu.async_copy(h_hbm.at[sidx.at[j]], rows.at[j], sem)
                   for j in range(NSUB)]
            for cp in cps:
                cp.wait()
            for j in range(NSUB):
                pltpu.sync_copy(rows.at[j], acc.at[didx.at[j]], add=True)
            return 0
        lax.fori_loop(0, nchunks, body, 0)

    @pl.when(cid == 0)
    def _():
        _edge_chunks(srcE, dstE, sid * EPT_B, EPT_B // CH)

    @pl.when(cid == 1)
    def _():
        _edge_chunks(srcV, dstV, sid * EVPT_B, EVPT_B // CH)

    plsc.subcore_barrier()

    out = [aggE_out, aggV_out]
    for c in range(NCORES):
        @pl.when(cid == c)
        def _():
            for k in range(NPT // ZR):
                slk = pl.ds(pl.multiple_of(sid * NPT + k * ZR, ZR), ZR)
                pltpu.sync_copy(acc.at[slk], zrow)
                pltpu.sync_copy(zrow, out[c].at[slk])


@functools.cache
def _sc_kernels():
    mesh = plsc.VectorSubcoreMesh(core_axis_name="c", subcore_axis_name="s")
    sc_params = pltpu.CompilerParams(use_tc_tiling_on_sc=False)
    scalar_segsum = pl.kernel(
        _scalar_segsum_body,
        out_type=[
            jax.ShapeDtypeStruct((NCORES, N_PAD), jnp.float32),
            jax.ShapeDtypeStruct((NCORES, N_PAD), jnp.float32),
        ],
        mesh=mesh,
        scratch_types=[
            pltpu.VMEM((NSUB, SUB), jnp.int32),
            pltpu.VMEM((NSUB, SUB), jnp.int32),
            pltpu.VMEM((NSUB, SUB), jnp.float32),
            pltpu.VMEM((NPT,), jnp.float32),
            pltpu.VMEM_SHARED((N_PAD,), jnp.float32),
            pltpu.VMEM_SHARED((N_PAD,), jnp.float32),
            pltpu.VMEM_SHARED((N_PAD,), jnp.float32),
            pltpu.SemaphoreType.DMA,
        ],
        compiler_params=sc_params,
    )
    row_segsum = pl.kernel(
        _row_segsum_body,
        out_type=[
            jax.ShapeDtypeStruct((N_PAD, H), jnp.float32),
            jax.ShapeDtypeStruct((N_PAD, H), jnp.float32),
        ],
        mesh=mesh,
        scratch_types=[
            pltpu.VMEM((NSUB, SUB), jnp.int32),
            pltpu.VMEM((NSUB, SUB), jnp.int32),
            pltpu.VMEM((NSUB, SUB, H), jnp.float32),
            pltpu.VMEM((ZR, H), jnp.float32),
            pltpu.VMEM_SHARED((N_PAD, H), jnp.float32),
            pltpu.SemaphoreType.DMA,
        ],
        compiler_params=sc_params,
    )
    return scalar_segsum, row_segsum


NB = 16
RB = N_PAD // NB


def _h_body(x_ref, se0_ref, se1_ref, sv0_ref, sv1_ref, l1_ref, h_ref):
    x = x_ref[0]
    se = se0_ref[0] + se1_ref[0]
    sv = sv0_ref[0] + sv1_ref[0]
    p = l1_ref[0:1, :]
    q = l1_ref[1:2, :]
    r = l1_ref[2:3, :]
    c = l1_ref[3:4, :]
    h_ref[...] = jnp.maximum(se * p + sv * q + x * r + c, 0.0)


def _dense_h(x_r, se0, se1, sv0, sv1, l1):
    col = pl.BlockSpec((1, RB, 1), lambda i: (i, 0, 0))
    return pl.pallas_call(
        _h_body,
        grid=(NB,),
        in_specs=[col, col, col, col, col,
                  pl.BlockSpec((8, H), lambda i: (0, 0))],
        out_specs=pl.BlockSpec((RB, H), lambda i: (i, 0)),
        out_shape=jax.ShapeDtypeStruct((N_PAD, H), jnp.float32),
    )(x_r, se0, se1, sv0, sv1, l1)


def _final_body(h_ref, ae_ref, av_ref, b_ref, wd_ref, wu_ref, wr_ref,
                b2_ref, wc_ref, bc_ref, out_ref, acc):
    i = pl.program_id(0)

    @pl.when(i == 0)
    def _():
        acc[...] = jnp.zeros((G, H), jnp.float32)

    h2 = jnp.maximum(
        jax.lax.dot_general(ae_ref[...], wd_ref[...], (((1,), (0,)), ((), ())))
        + jax.lax.dot_general(av_ref[...], wu_ref[...], (((1,), (0,)), ((), ())))
        + jax.lax.dot_general(h_ref[...], wr_ref[...], (((1,), (0,)), ((), ())))
        + b2_ref[0:1, :], 0.0)
    bcol = b_ref[0]
    iot = lax.broadcasted_iota(jnp.int32, (1, G), 1)
    onehot = (bcol == iot).astype(jnp.float32)
    acc[...] += jax.lax.dot_general(onehot, h2, (((0,), (0,)), ((), ())))

    @pl.when(i == NB - 1)
    def _():
        out_ref[...] = (
            jax.lax.dot_general(acc[...], wc_ref[...], (((1,), (0,)), ((), ())))
            + bc_ref[0:1, :])


def _dense_final(h, agg_e, agg_v, batch_r, wd, wu, wr, b2, wc_pad, bc_pad):
    row = pl.BlockSpec((RB, H), lambda i: (i, 0))
    w16 = pl.BlockSpec((H, H), lambda i: (0, 0))
    return pl.pallas_call(
        _final_body,
        grid=(NB,),
        in_specs=[row, row, row,
                  pl.BlockSpec((1, RB, 1), lambda i: (i, 0, 0)),
                  w16, w16, w16,
                  pl.BlockSpec((8, H), lambda i: (0, 0)),
                  pl.BlockSpec((H, 128), lambda i: (0, 0)),
                  pl.BlockSpec((8, 128), lambda i: (0, 0))],
        out_specs=pl.BlockSpec((G, 128), lambda i: (0, 0)),
        out_shape=jax.ShapeDtypeStruct((G, 128), jnp.float32),
        scratch_shapes=[pltpu.VMEM((G, H), jnp.float32)],
    )(h, agg_e, agg_v, batch_r, wd, wu, wr, b2, wc_pad, bc_pad)


def _pad_edges(ei, ep):
    pe = ep - ei.shape[1]
    i = jnp.arange(pe, dtype=jnp.int32)
    src = jnp.concatenate([ei[0], i % N])
    dst = jnp.concatenate([ei[1], N + (i % (N_PAD - N))])
    return src.reshape(ep // SUB, SUB), dst.reshape(ep // SUB, SUB)


def kernel(x, edge_index, v_edge_index, batch,
           W_rel_d1, b_rel_d1, W_root_d1,
           W_rel_u1, b_rel_u1, W_root_u1,
           W_rel_d2, b_rel_d2, W_root_d2,
           W_rel_u2, b_rel_u2, W_root_u2,
           W_cls, b_cls):
    xf = x[:, 0]
    x_pad = jnp.pad(xf, (0, N_PAD - N))
    srcE, dstE = _pad_edges(edge_index, E_PAD)
    srcV, dstV = _pad_edges(v_edge_index, EV_PAD)

    scalar_segsum, row_segsum = _sc_kernels()
    se_part, sv_part = scalar_segsum(x_pad, srcE, dstE, srcV, dstV)

    l1 = jnp.zeros((8, H), jnp.float32)
    l1 = l1.at[0].set(W_rel_d1[0]).at[1].set(W_rel_u1[0])
    l1 = l1.at[2].set(W_root_d1[0] + W_root_u1[0])
    l1 = l1.at[3].set(b_rel_d1 + b_rel_u1)

    rs = lambda a: a.reshape(NB, RB, 1)
    h = _dense_h(rs(x_pad), rs(se_part[0]), rs(se_part[1]),
                 rs(sv_part[0]), rs(sv_part[1]), l1)

    agg_e, agg_v = row_segsum(h, srcE, dstE, srcV, dstV)

    batch_pad = jnp.pad(batch, (0, N_PAD - N), constant_values=G)
    b2 = jnp.zeros((8, H), jnp.float32).at[0].set(b_rel_d2 + b_rel_u2)
    wc_pad = jnp.zeros((H, 128), jnp.float32).at[:, :C].set(W_cls)
    bc_pad = jnp.zeros((8, 128), jnp.float32).at[0, :C].set(b_cls)
    out = _dense_final(h, agg_e, agg_v, batch_pad.reshape(NB, RB, 1),
                       W_rel_d2, W_rel_u2, W_root_d2 + W_root_u2,
                       b2, wc_pad, bc_pad)
    return out[:, :C]


# speedup vs baseline: 69.8878x; 1.1896x over previous
import functools

import jax
import jax.numpy as jnp
from jax import lax
from jax.experimental import pallas as pl
from jax.experimental.pallas import tpu as pltpu
from jax.experimental.pallas import tpu_sc as plsc

N = 50000
E = 3200000
EV = 1600000
G = 128
H = 16
C = 2

NCORES = 2
NTILES = 16
NPT = 3200
N_PAD = NTILES * NPT
SUB = 128
ZR = 400

CH_A = 2048
NSUB_A = CH_A // SUB
CH_B = 1024
NSUB_B = CH_B // SUB

E_PAD = 32 * 50 * CH_A
EV_PAD = 32 * 26 * CH_A
EPT = E_PAD // 32
EVPT = EV_PAD // 32


def _pipe(src_hbm, dst_hbm, gtab, acc, dummy_hbm,
          sidx, didx, vals, gsem, ssem, edge_off, nchunks, nsub):
    ch = nsub * SUB
    nch2 = nchunks // 2

    def stage(g, b):
        row0 = pl.multiple_of((edge_off + g * ch) // SUB, nsub)
        pltpu.sync_copy(src_hbm.at[pl.ds(row0, nsub)], sidx[b])
        pltpu.sync_copy(dst_hbm.at[pl.ds(row0, nsub)], didx[b])

    def fire_gathers(b):
        for j in range(nsub):
            pltpu.async_copy(gtab.at[sidx[b].at[j]], vals[b].at[j], gsem[b])

    def fire_scatters(b):
        for j in range(nsub):
            pltpu.async_copy(vals[b].at[j], acc.at[didx[b].at[j]],
                             ssem[b], add=True)

    def drain(sem, b):
        for j in range(nsub):
            pltpu.make_async_copy(
                dummy_hbm.at[pl.ds(0, SUB)], vals[b].at[j], sem).wait()

    stage(0, 0)
    fire_gathers(0)

    def pair(i, _):
        drain(gsem[0], 0)
        fire_scatters(0)

        @pl.when(i > 0)
        def _():
            drain(ssem[1], 1)
        stage(2 * i + 1, 1)
        fire_gathers(1)

        drain(gsem[1], 1)
        fire_scatters(1)

        @pl.when(i < nch2 - 1)
        def _():
            drain(ssem[0], 0)
            stage(2 * i + 2, 0)
            fire_gathers(0)
        return 0

    lax.fori_loop(0, nch2, pair, 0)
    drain(ssem[0], 0)
    drain(ssem[1], 1)


def _scalar_segsum_body(x_hbm, srcE, dstE, srcV, dstV, se_out, sv_out,
                        sidx0, sidx1, didx0, didx1, vals0, vals1, zbuf,
                        x_sp, acc_e, acc_v, gsem0, gsem1, ssem0, ssem1):
    cid = lax.axis_index("c")
    sid = lax.axis_index("s")

    def _zero(i, _):
        zbuf[pl.ds(pl.multiple_of(i * 16, 16), 16)] = jnp.zeros((16,), jnp.float32)
        return 0
    lax.fori_loop(0, NPT // 16, _zero, 0)
    sl = pl.ds(pl.multiple_of(sid * NPT, NPT), NPT)
    pltpu.sync_copy(zbuf, acc_e.at[sl])
    pltpu.sync_copy(zbuf, acc_v.at[sl])
    pltpu.sync_copy(x_hbm.at[sl], zbuf)
    pltpu.sync_copy(zbuf, x_sp.at[sl])
    plsc.subcore_barrier()

    tile = cid * NTILES + sid
    sidx = (sidx0, sidx1)
    didx = (didx0, didx1)
    vals = (vals0, vals1)
    gsem = (gsem0, gsem1)
    ssem = (ssem0, ssem1)
    _pipe(srcE, dstE, x_sp, acc_e, x_hbm, sidx, didx, vals, gsem, ssem,
          tile * EPT, EPT // CH_A, NSUB_A)
    _pipe(srcV, dstV, x_sp, acc_v, x_hbm, sidx, didx, vals, gsem, ssem,
          tile * EVPT, EVPT // CH_A, NSUB_A)
    plsc.subcore_barrier()

    pltpu.sync_copy(acc_e.at[sl], zbuf)
    pltpu.sync_copy(zbuf, se_out.at[cid].at[sl])
    pltpu.sync_copy(acc_v.at[sl], zbuf)
    pltpu.sync_copy(zbuf, sv_out.at[cid].at[sl])


def _row_segsum_body(hd_hbm, hu_hbm, srcE, dstE, srcV, dstV, accP_out,
                     sidx0, sidx1, didx0, didx1, rows0, rows1, zrow,
                     acc, gsem0, gsem1, ssem0, ssem1):
    cid = lax.axis_index("c")
    sid = lax.axis_index("s")

    def _zero(i, _):
        zrow[i, :] = jnp.zeros((16,), jnp.float32)
        return 0
    lax.fori_loop(0, ZR, _zero, 0)
    for k in range(NPT // ZR):
        pltpu.sync_copy(
            zrow, acc.at[pl.ds(pl.multiple_of(sid * NPT + k * ZR, ZR), ZR)])
    plsc.subcore_barrier()

    sidx = (sidx0, sidx1)
    didx = (didx0, didx1)
    rows = (rows0, rows1)
    gsem = (gsem0, gsem1)
    ssem = (ssem0, ssem1)
    _pipe(srcE, dstE, hd_hbm, acc, hd_hbm, sidx, didx, rows, gsem, ssem,
          cid * (E_PAD // 2) + sid * EPT, EPT // CH_B, NSUB_B)
    _pipe(srcV, dstV, hu_hbm, acc, hu_hbm, sidx, didx, rows, gsem, ssem,
          cid * (EV_PAD // 2) + sid * EVPT, EVPT // CH_B, NSUB_B)
    plsc.subcore_barrier()

    for k in range(NPT // ZR):
        slk = pl.ds(pl.multiple_of(sid * NPT + k * ZR, ZR), ZR)
        pltpu.sync_copy(acc.at[slk], zrow)
        pltpu.sync_copy(zrow, accP_out.at[cid].at[slk])


@functools.cache
def _sc_kernels():
    mesh = plsc.VectorSubcoreMesh(core_axis_name="c", subcore_axis_name="s")
    sc_params = pltpu.CompilerParams(use_tc_tiling_on_sc=False)
    scalar_segsum = pl.kernel(
        _scalar_segsum_body,
        out_type=[
            jax.ShapeDtypeStruct((NCORES, N_PAD), jnp.float32),
            jax.ShapeDtypeStruct((NCORES, N_PAD), jnp.float32),
        ],
        mesh=mesh,
        scratch_types=[
            pltpu.VMEM((NSUB_A, SUB), jnp.int32),
            pltpu.VMEM((NSUB_A, SUB), jnp.int32),
            pltpu.VMEM((NSUB_A, SUB), jnp.int32),
            pltpu.VMEM((NSUB_A, SUB), jnp.int32),
            pltpu.VMEM((NSUB_A, SUB), jnp.float32),
            pltpu.VMEM((NSUB_A, SUB), jnp.float32),
            pltpu.VMEM((NPT,), jnp.float32),
            pltpu.VMEM_SHARED((N_PAD,), jnp.float32),
            pltpu.VMEM_SHARED((N_PAD,), jnp.float32),
            pltpu.VMEM_SHARED((N_PAD,), jnp.float32),
            pltpu.SemaphoreType.DMA,
            pltpu.SemaphoreType.DMA,
            pltpu.SemaphoreType.DMA,
            pltpu.SemaphoreType.DMA,
        ],
        compiler_params=sc_params,
    )
    row_segsum = pl.kernel(
        _row_segsum_body,
        out_type=jax.ShapeDtypeStruct((NCORES, N_PAD, H), jnp.float32),
        mesh=mesh,
        scratch_types=[
            pltpu.VMEM((NSUB_B, SUB), jnp.int32),
            pltpu.VMEM((NSUB_B, SUB), jnp.int32),
            pltpu.VMEM((NSUB_B, SUB), jnp.int32),
            pltpu.VMEM((NSUB_B, SUB), jnp.int32),
            pltpu.VMEM((NSUB_B, SUB, H), jnp.float32),
            pltpu.VMEM((NSUB_B, SUB, H), jnp.float32),
            pltpu.VMEM((ZR, H), jnp.float32),
            pltpu.VMEM_SHARED((N_PAD, H), jnp.float32),
            pltpu.SemaphoreType.DMA,
            pltpu.SemaphoreType.DMA,
            pltpu.SemaphoreType.DMA,
            pltpu.SemaphoreType.DMA,
        ],
        compiler_params=sc_params,
    )
    return scalar_segsum, row_segsum


NB = 16
RB = N_PAD // NB


def _h_body(x_ref, se0_ref, se1_ref, sv0_ref, sv1_ref, l1_ref,
            wd_ref, wu_ref, h_ref, hd_ref, hu_ref):
    x = x_ref[0]
    se = se0_ref[0] + se1_ref[0]
    sv = sv0_ref[0] + sv1_ref[0]
    p = l1_ref[0:1, :]
    q = l1_ref[1:2, :]
    r = l1_ref[2:3, :]
    c = l1_ref[3:4, :]
    h = jnp.maximum(se * p + sv * q + x * r + c, 0.0)
    h_ref[...] = h
    hd_ref[...] = jax.lax.dot_general(h, wd_ref[...], (((1,), (0,)), ((), ())))
    hu_ref[...] = jax.lax.dot_general(h, wu_ref[...], (((1,), (0,)), ((), ())))


def _dense_h(x_r, se0, se1, sv0, sv1, l1, wd, wu):
    col = pl.BlockSpec((1, RB, 1), lambda i: (i, 0, 0))
    w16 = pl.BlockSpec((H, H), lambda i: (0, 0))
    row = pl.BlockSpec((RB, H), lambda i: (i, 0))
    out = jax.ShapeDtypeStruct((N_PAD, H), jnp.float32)
    return pl.pallas_call(
        _h_body,
        grid=(NB,),
        in_specs=[col, col, col, col, col,
                  pl.BlockSpec((8, H), lambda i: (0, 0)), w16, w16],
        out_specs=[row, row, row],
        out_shape=[out, out, out],
    )(x_r, se0, se1, sv0, sv1, l1, wd, wu)


def _final_body(h_ref, a0_ref, a1_ref, b_ref, wr_ref,
                b2_ref, wc_ref, bc_ref, out_ref, acc):
    i = pl.program_id(0)

    @pl.when(i == 0)
    def _():
        acc[...] = jnp.zeros((G, H), jnp.float32)

    h2 = jnp.maximum(
        a0_ref[...] + a1_ref[...]
        + jax.lax.dot_general(h_ref[...], wr_ref[...], (((1,), (0,)), ((), ())))
        + b2_ref[0:1, :], 0.0)
    bcol = b_ref[0]
    iot = lax.broadcasted_iota(jnp.int32, (1, G), 1)
    onehot = (bcol == iot).astype(jnp.float32)
    acc[...] += jax.lax.dot_general(onehot, h2, (((0,), (0,)), ((), ())))

    @pl.when(i == NB - 1)
    def _():
        out_ref[...] = (
            jax.lax.dot_general(acc[...], wc_ref[...], (((1,), (0,)), ((), ())))
            + bc_ref[0:1, :])


def _dense_final(h, a0, a1, batch_r, wr, b2, wc_pad, bc_pad):
    row = pl.BlockSpec((RB, H), lambda i: (i, 0))
    return pl.pallas_call(
        _final_body,
        grid=(NB,),
        in_specs=[row, row, row,
                  pl.BlockSpec((1, RB, 1), lambda i: (i, 0, 0)),
                  pl.BlockSpec((H, H), lambda i: (0, 0)),
                  pl.BlockSpec((8, H), lambda i: (0, 0)),
                  pl.BlockSpec((H, 128), lambda i: (0, 0)),
                  pl.BlockSpec((8, 128), lambda i: (0, 0))],
        out_specs=pl.BlockSpec((G, 128), lambda i: (0, 0)),
        out_shape=jax.ShapeDtypeStruct((G, 128), jnp.float32),
        scratch_shapes=[pltpu.VMEM((G, H), jnp.float32)],
    )(h, a0, a1, batch_r, wr, b2, wc_pad, bc_pad)


def _pad_edges(ei, ep):
    pe = ep - ei.shape[1]
    i = jnp.arange(pe, dtype=jnp.int32)
    src = jnp.concatenate([ei[0], i % N])
    dst = jnp.concatenate([ei[1], N + (i % (N_PAD - N))])
    return src.reshape(ep // SUB, SUB), dst.reshape(ep // SUB, SUB)


def kernel(x, edge_index, v_edge_index, batch,
           W_rel_d1, b_rel_d1, W_root_d1,
           W_rel_u1, b_rel_u1, W_root_u1,
           W_rel_d2, b_rel_d2, W_root_d2,
           W_rel_u2, b_rel_u2, W_root_u2,
           W_cls, b_cls):
    xf = x[:, 0]
    x_pad = jnp.pad(xf, (0, N_PAD - N))
    srcE, dstE = _pad_edges(edge_index, E_PAD)
    srcV, dstV = _pad_edges(v_edge_index, EV_PAD)

    scalar_segsum, row_segsum = _sc_kernels()
    se_part, sv_part = scalar_segsum(x_pad, srcE, dstE, srcV, dstV)

    l1 = jnp.zeros((8, H), jnp.float32)
    l1 = l1.at[0].set(W_rel_d1[0]).at[1].set(W_rel_u1[0])
    l1 = l1.at[2].set(W_root_d1[0] + W_root_u1[0])
    l1 = l1.at[3].set(b_rel_d1 + b_rel_u1)

    rs = lambda a: a.reshape(NB, RB, 1)
    h, hd, hu = _dense_h(rs(x_pad), rs(se_part[0]), rs(se_part[1]),
                         rs(sv_part[0]), rs(sv_part[1]), l1,
                         W_rel_d2, W_rel_u2)

    accP = row_segsum(hd, hu, srcE, dstE, srcV, dstV)

    batch_pad = jnp.pad(batch, (0, N_PAD - N), constant_values=G)
    b2 = jnp.zeros((8, H), jnp.float32).at[0].set(b_rel_d2 + b_rel_u2)
    wc_pad = jnp.zeros((H, 128), jnp.float32).at[:, :C].set(W_cls)
    bc_pad = jnp.zeros((8, 128), jnp.float32).at[0, :C].set(b_cls)
    out = _dense_final(h, accP[0], accP[1], batch_pad.reshape(NB, RB, 1),
                       W_root_d2 + W_root_u2, b2, wc_pad, bc_pad)
    return out[:, :C]


# speedup vs baseline: 85.2120x; 1.2193x over previous
import functools

import jax
import jax.numpy as jnp
import numpy as np
from jax import lax
from jax.experimental import pallas as pl
from jax.experimental.pallas import tpu as pltpu
from jax.experimental.pallas import tpu_sc as plsc

N = 50000
E = 3200000
EV = 1600000
G = 128
H = 16
C = 2

NCORES = 2
NTILES = 16
NPT = 3200
N_PAD = NTILES * NPT
SUB = 128
ZR = 400

CH_A = 2048
NSUB_A = CH_A // SUB
CH_B = 1024
NSUB_B = CH_B // SUB

E_PAD = 32 * 50 * CH_A
EV_PAD = 32 * 26 * CH_A
EPT = E_PAD // 32
EVPT = EV_PAD // 32


def _pad_const(ep, e):
    pe = ep - e
    i = np.arange(pe, dtype=np.int64)
    src = (i % N).astype(np.int32).reshape(pe // SUB, SUB)
    dst = (N + i % (N_PAD - N)).astype(np.int32).reshape(pe // SUB, SUB)
    return src, dst


_SRC_E_PAD, _DST_E_PAD = _pad_const(E_PAD, E)
_SRC_V_PAD, _DST_V_PAD = _pad_const(EV_PAD, EV)


def _pipe(src_hbm, dst_hbm, gtab, acc, dummy_hbm,
          sidx, didx, vals, gsem, ssem, edge_off, nchunks, nsub):
    ch = nsub * SUB
    nch2 = nchunks // 2

    def stage(g, b):
        row0 = pl.multiple_of((edge_off + g * ch) // SUB, nsub)
        pltpu.sync_copy(src_hbm.at[pl.ds(row0, nsub)], sidx[b])
        pltpu.sync_copy(dst_hbm.at[pl.ds(row0, nsub)], didx[b])

    def fire_gathers(b):
        for j in range(nsub):
            pltpu.async_copy(gtab.at[sidx[b].at[j]], vals[b].at[j], gsem[b])

    def fire_scatters(b):
        for j in range(nsub):
            pltpu.async_copy(vals[b].at[j], acc.at[didx[b].at[j]],
                             ssem[b], add=True)

    def drain(sem, b):
        for j in range(nsub):
            pltpu.make_async_copy(
                dummy_hbm.at[pl.ds(0, SUB)], vals[b].at[j], sem).wait()

    stage(0, 0)
    fire_gathers(0)

    def pair(i, _):
        drain(gsem[0], 0)
        fire_scatters(0)

        @pl.when(i > 0)
        def _():
            drain(ssem[1], 1)
        stage(2 * i + 1, 1)
        fire_gathers(1)

        drain(gsem[1], 1)
        fire_scatters(1)

        @pl.when(i < nch2 - 1)
        def _():
            drain(ssem[0], 0)
            stage(2 * i + 2, 0)
            fire_gathers(0)
        return 0

    lax.fori_loop(0, nch2, pair, 0)
    drain(ssem[0], 0)
    drain(ssem[1], 1)


def _scalar_segsum_body(x_hbm, b_hbm, srcE, dstE, srcV, dstV, comb_out,
                        sidx0, sidx1, didx0, didx1, vals0, vals1, zbuf,
                        x_sp, acc_e, acc_v, gsem0, gsem1, ssem0, ssem1):
    cid = lax.axis_index("c")
    sid = lax.axis_index("s")

    def _zero(i, _):
        zbuf[pl.ds(pl.multiple_of(i * 16, 16), 16)] = jnp.zeros((16,), jnp.float32)
        return 0
    lax.fori_loop(0, NPT // 16, _zero, 0)
    sl = pl.ds(pl.multiple_of(sid * NPT, NPT), NPT)
    pltpu.sync_copy(zbuf, acc_e.at[sl])
    pltpu.sync_copy(zbuf, acc_v.at[sl])
    pltpu.sync_copy(x_hbm.at[sl], zbuf)
    pltpu.sync_copy(zbuf, x_sp.at[sl])

    @pl.when(cid == 0)
    def _():
        pltpu.sync_copy(zbuf, comb_out.at[4].at[sl])
        pltpu.sync_copy(b_hbm.at[sl], zbuf)
        pltpu.sync_copy(zbuf, comb_out.at[5].at[sl])
    plsc.subcore_barrier()

    tile = cid * NTILES + sid
    sidx = (sidx0, sidx1)
    didx = (didx0, didx1)
    vals = (vals0, vals1)
    gsem = (gsem0, gsem1)
    ssem = (ssem0, ssem1)
    _pipe(srcE, dstE, x_sp, acc_e, x_hbm, sidx, didx, vals, gsem, ssem,
          tile * EPT, EPT // CH_A, NSUB_A)
    _pipe(srcV, dstV, x_sp, acc_v, x_hbm, sidx, didx, vals, gsem, ssem,
          tile * EVPT, EVPT // CH_A, NSUB_A)
    plsc.subcore_barrier()

    pltpu.sync_copy(acc_e.at[sl], zbuf)
    pltpu.sync_copy(zbuf, comb_out.at[cid].at[sl])
    pltpu.sync_copy(acc_v.at[sl], zbuf)
    pltpu.sync_copy(zbuf, comb_out.at[cid + 2].at[sl])


def _row_segsum_body(hd_hbm, hu_hbm, srcE, dstE, srcV, dstV, accP_out,
                     sidx0, sidx1, didx0, didx1, rows0, rows1, zrow,
                     acc, gsem0, gsem1, ssem0, ssem1):
    cid = lax.axis_index("c")
    sid = lax.axis_index("s")

    def _zero(i, _):
        zrow[i, :] = jnp.zeros((16,), jnp.float32)
        return 0
    lax.fori_loop(0, ZR, _zero, 0)
    for k in range(NPT // ZR):
        pltpu.sync_copy(
            zrow, acc.at[pl.ds(pl.multiple_of(sid * NPT + k * ZR, ZR), ZR)])
    plsc.subcore_barrier()

    sidx = (sidx0, sidx1)
    didx = (didx0, didx1)
    rows = (rows0, rows1)
    gsem = (gsem0, gsem1)
    ssem = (ssem0, ssem1)
    _pipe(srcE, dstE, hd_hbm, acc, hd_hbm, sidx, didx, rows, gsem, ssem,
          cid * (E_PAD // 2) + sid * EPT, EPT // CH_B, NSUB_B)
    _pipe(srcV, dstV, hu_hbm, acc, hu_hbm, sidx, didx, rows, gsem, ssem,
          cid * (EV_PAD // 2) + sid * EVPT, EVPT // CH_B, NSUB_B)
    plsc.subcore_barrier()

    for k in range(NPT // ZR):
        slk = pl.ds(pl.multiple_of(sid * NPT + k * ZR, ZR), ZR)
        pltpu.sync_copy(acc.at[slk], zrow)
        pltpu.sync_copy(zrow, accP_out.at[cid].at[slk])


@functools.cache
def _sc_kernels():
    mesh = plsc.VectorSubcoreMesh(core_axis_name="c", subcore_axis_name="s")
    sc_params = pltpu.CompilerParams(use_tc_tiling_on_sc=False)
    scalar_segsum = pl.kernel(
        _scalar_segsum_body,
        out_type=jax.ShapeDtypeStruct((8, N_PAD), jnp.float32),
        mesh=mesh,
        scratch_types=[
            pltpu.VMEM((NSUB_A, SUB), jnp.int32),
            pltpu.VMEM((NSUB_A, SUB), jnp.int32),
            pltpu.VMEM((NSUB_A, SUB), jnp.int32),
            pltpu.VMEM((NSUB_A, SUB), jnp.int32),
            pltpu.VMEM((NSUB_A, SUB), jnp.float32),
            pltpu.VMEM((NSUB_A, SUB), jnp.float32),
            pltpu.VMEM((NPT,), jnp.float32),
            pltpu.VMEM_SHARED((N_PAD,), jnp.float32),
            pltpu.VMEM_SHARED((N_PAD,), jnp.float32),
            pltpu.VMEM_SHARED((N_PAD,), jnp.float32),
            pltpu.SemaphoreType.DMA,
            pltpu.SemaphoreType.DMA,
            pltpu.SemaphoreType.DMA,
            pltpu.SemaphoreType.DMA,
        ],
        compiler_params=sc_params,
    )
    row_segsum = pl.kernel(
        _row_segsum_body,
        out_type=jax.ShapeDtypeStruct((NCORES, N_PAD, H), jnp.float32),
        mesh=mesh,
        scratch_types=[
            pltpu.VMEM((NSUB_B, SUB), jnp.int32),
            pltpu.VMEM((NSUB_B, SUB), jnp.int32),
            pltpu.VMEM((NSUB_B, SUB), jnp.int32),
            pltpu.VMEM((NSUB_B, SUB), jnp.int32),
            pltpu.VMEM((NSUB_B, SUB, H), jnp.float32),
            pltpu.VMEM((NSUB_B, SUB, H), jnp.float32),
            pltpu.VMEM((ZR, H), jnp.float32),
            pltpu.VMEM_SHARED((N_PAD, H), jnp.float32),
            pltpu.SemaphoreType.DMA,
            pltpu.SemaphoreType.DMA,
            pltpu.SemaphoreType.DMA,
            pltpu.SemaphoreType.DMA,
        ],
        compiler_params=sc_params,
    )
    return scalar_segsum, row_segsum


NB = 16
RB = N_PAD // NB


def _h_body(comb_ref, l1t_ref, wd_ref, wu_ref, ht_ref, hd_ref, hu_ref):
    se = comb_ref[0:1, :] + comb_ref[1:2, :]
    sv = comb_ref[2:3, :] + comb_ref[3:4, :]
    xr = comb_ref[4:5, :]
    p = l1t_ref[:, 0:1]
    q = l1t_ref[:, 1:2]
    r = l1t_ref[:, 2:3]
    c = l1t_ref[:, 3:4]
    ht = jnp.maximum(p * se + q * sv + r * xr + c, 0.0)
    ht_ref[...] = ht
    hd_ref[...] = jax.lax.dot_general(ht, wd_ref[...], (((0,), (0,)), ((), ())))
    hu_ref[...] = jax.lax.dot_general(ht, wu_ref[...], (((0,), (0,)), ((), ())))


def _dense_h(comb, l1t, wd, wu):
    w16 = pl.BlockSpec((H, H), lambda i: (0, 0))
    rowspec = pl.BlockSpec((RB, H), lambda i: (i, 0))
    return pl.pallas_call(
        _h_body,
        grid=(NB,),
        in_specs=[pl.BlockSpec((8, RB), lambda i: (0, i)),
                  pl.BlockSpec((H, 8), lambda i: (0, 0)), w16, w16],
        out_specs=[pl.BlockSpec((H, RB), lambda i: (0, i)), rowspec, rowspec],
        out_shape=[jax.ShapeDtypeStruct((H, N_PAD), jnp.float32),
                   jax.ShapeDtypeStruct((N_PAD, H), jnp.float32),
                   jax.ShapeDtypeStruct((N_PAD, H), jnp.float32)],
    )(comb, l1t, wd, wu)


def _final_body(ht_ref, a0_ref, a1_ref, comb_ref, wr_ref,
                b2_ref, wc_ref, bc_ref, out_ref, acc):
    i = pl.program_id(0)

    @pl.when(i == 0)
    def _():
        acc[...] = jnp.zeros((G, H), jnp.float32)

    h2 = jnp.maximum(
        a0_ref[0] + a1_ref[0]
        + jax.lax.dot_general(ht_ref[...], wr_ref[...], (((0,), (0,)), ((), ())))
        + b2_ref[0:1, :], 0.0)
    bi = lax.bitcast_convert_type(comb_ref[5:6, :], jnp.int32)
    iot = lax.broadcasted_iota(jnp.int32, (G, 1), 0)
    onehot = (bi == iot).astype(jnp.float32)
    acc[...] += jax.lax.dot_general(onehot, h2, (((1,), (0,)), ((), ())))

    @pl.when(i == NB - 1)
    def _():
        out_ref[...] = (
            jax.lax.dot_general(acc[...], wc_ref[...], (((1,), (0,)), ((), ())))
            + bc_ref[0:1, :])


def _dense_final(ht, accP, comb, wr, b2, wc_pad, bc_pad):
    return pl.pallas_call(
        _final_body,
        grid=(NB,),
        in_specs=[pl.BlockSpec((H, RB), lambda i: (0, i)),
                  pl.BlockSpec((1, RB, H), lambda i: (0, i, 0)),
                  pl.BlockSpec((1, RB, H), lambda i: (1, i, 0)),
                  pl.BlockSpec((8, RB), lambda i: (0, i)),
                  pl.BlockSpec((H, H), lambda i: (0, 0)),
                  pl.BlockSpec((8, H), lambda i: (0, 0)),
                  pl.BlockSpec((H, 128), lambda i: (0, 0)),
                  pl.BlockSpec((8, 128), lambda i: (0, 0))],
        out_specs=pl.BlockSpec((G, 128), lambda i: (0, 0)),
        out_shape=jax.ShapeDtypeStruct((G, 128), jnp.float32),
        scratch_shapes=[pltpu.VMEM((G, H), jnp.float32)],
    )(ht, accP, accP, comb, wr, b2, wc_pad, bc_pad)


def kernel(x, edge_index, v_edge_index, batch,
           W_rel_d1, b_rel_d1, W_root_d1,
           W_rel_u1, b_rel_u1, W_root_u1,
           W_rel_d2, b_rel_d2, W_root_d2,
           W_rel_u2, b_rel_u2, W_root_u2,
           W_cls, b_cls):
    x_pad = jnp.pad(x[:, 0], (0, N_PAD - N))
    batch_bits = lax.bitcast_convert_type(
        jnp.pad(batch, (0, N_PAD - N), constant_values=G), jnp.float32)
    srcE = jnp.concatenate(
        [edge_index[0].reshape(E // SUB, SUB), jnp.asarray(_SRC_E_PAD)], axis=0)
    dstE = jnp.concatenate(
        [edge_index[1].reshape(E // SUB, SUB), jnp.asarray(_DST_E_PAD)], axis=0)
    srcV = jnp.concatenate(
        [v_edge_index[0].reshape(EV // SUB, SUB), jnp.asarray(_SRC_V_PAD)], axis=0)
    dstV = jnp.concatenate(
        [v_edge_index[1].reshape(EV // SUB, SUB), jnp.asarray(_DST_V_PAD)], axis=0)

    scalar_segsum, row_segsum = _sc_kernels()
    comb = scalar_segsum(x_pad, batch_bits, srcE, dstE, srcV, dstV)

    l1t = jnp.zeros((H, 8), jnp.float32)
    l1t = l1t.at[:, 0].set(W_rel_d1[0]).at[:, 1].set(W_rel_u1[0])
    l1t = l1t.at[:, 2].set(W_root_d1[0] + W_root_u1[0])
    l1t = l1t.at[:, 3].set(b_rel_d1 + b_rel_u1)

    ht, hd, hu = _dense_h(comb, l1t, W_rel_d2, W_rel_u2)

    accP = row_segsum(hd, hu, srcE, dstE, srcV, dstV)

    b2 = jnp.zeros((8, H), jnp.float32).at[0].set(b_rel_d2 + b_rel_u2)
    wc_pad = jnp.zeros((H, 128), jnp.float32).at[:, :C].set(W_cls)
    bc_pad = jnp.zeros((8, 128), jnp.float32).at[0, :C].set(b_cls)
    out = _dense_final(ht, accP, comb, W_root_d2 + W_root_u2,
                       b2, wc_pad, bc_pad)
    return out[:, :C]


# speedup vs baseline: 110.1573x; 1.2927x over previous
import functools

import jax
import jax.numpy as jnp
import numpy as np
from jax import lax
from jax.experimental import pallas as pl
from jax.experimental.pallas import tpu as pltpu
from jax.experimental.pallas import tpu_sc as plsc

N = 50000
E = 3200000
EV = 1600000
G = 128
H = 16
C = 2

NCORES = 2
NTILES = 16
NPT = 3200
N_PAD = NTILES * NPT
SUB = 128
ZR = 400

CH_A = 2048
NSUB_A = CH_A // SUB
CH_B = 1024
NSUB_B = CH_B // SUB

E_PAD = 32 * 50 * CH_A
EV_PAD = 32 * 26 * CH_A
EPT = E_PAD // 32
EVPT = EV_PAD // 32


def _pad_const(ep, e):
    pe = ep - e
    i = np.arange(pe, dtype=np.int64)
    src = (i % N).astype(np.int32)
    dst = (N + i % (N_PAD - N)).astype(np.int32)
    return src, dst


_SRC_E_PAD, _DST_E_PAD = _pad_const(E_PAD, E)
_SRC_V_PAD, _DST_V_PAD = _pad_const(EV_PAD, EV)


def _pipe(src_hbm, dst_hbm, gtab, acc, dummy_hbm,
          sidx, didx, vals, gsem, ssem, tsem, edge_off, nchunks, nsub):
    ch = nsub * SUB
    nch2 = nchunks // 2

    def stage(g, b):
        for j in range(nsub):
            e0 = pl.multiple_of(edge_off + g * ch + j * SUB, SUB)
            pltpu.async_copy(src_hbm.at[pl.ds(e0, SUB)], sidx[b].at[j], tsem[b])
            pltpu.async_copy(dst_hbm.at[pl.ds(e0, SUB)], didx[b].at[j], tsem[b])

    def drain_stage(b):
        for j in range(2 * nsub):
            pltpu.make_async_copy(
                src_hbm.at[pl.ds(0, SUB)], sidx[b].at[0], tsem[b]).wait()

    def fire_gathers(b):
        for j in range(nsub):
            pltpu.async_copy(gtab.at[sidx[b].at[j]], vals[b].at[j], gsem[b])

    def fire_scatters(b):
        for j in range(nsub):
            pltpu.async_copy(vals[b].at[j], acc.at[didx[b].at[j]],
                             ssem[b], add=True)

    def drain(sem, b):
        for j in range(nsub):
            pltpu.make_async_copy(
                dummy_hbm.at[pl.ds(0, SUB)], vals[b].at[j], sem).wait()

    stage(0, 0)
    drain_stage(0)
    fire_gathers(0)

    def pair(i, _):
        @pl.when(i > 0)
        def _():
            drain(ssem[1], 1)
        stage(2 * i + 1, 1)
        drain(gsem[0], 0)
        fire_scatters(0)
        drain_stage(1)
        fire_gathers(1)

        @pl.when(i < nch2 - 1)
        def _():
            drain(ssem[0], 0)
            stage(2 * i + 2, 0)
        drain(gsem[1], 1)
        fire_scatters(1)

        @pl.when(i < nch2 - 1)
        def _():
            drain_stage(0)
            fire_gathers(0)
        return 0

    lax.fori_loop(0, nch2, pair, 0)
    drain(ssem[0], 0)
    drain(ssem[1], 1)


def _scalar_segsum_body(x_hbm, b_hbm, srcE, dstE, srcV, dstV, comb_out,
                        sidx0, sidx1, didx0, didx1, vals0, vals1, zbuf,
                        x_sp, acc_e, acc_v, gsem0, gsem1, ssem0, ssem1,
                        tsem0, tsem1):
    cid = lax.axis_index("c")
    sid = lax.axis_index("s")

    def _zero(i, _):
        zbuf[pl.ds(pl.multiple_of(i * 16, 16), 16)] = jnp.zeros((16,), jnp.float32)
        return 0
    lax.fori_loop(0, NPT // 16, _zero, 0)
    sl = pl.ds(pl.multiple_of(sid * NPT, NPT), NPT)
    pltpu.sync_copy(zbuf, acc_e.at[sl])
    pltpu.sync_copy(zbuf, acc_v.at[sl])
    pltpu.sync_copy(x_hbm.at[sl], zbuf)
    pltpu.sync_copy(zbuf, x_sp.at[sl])

    @pl.when(cid == 0)
    def _():
        pltpu.sync_copy(zbuf, comb_out.at[4].at[sl])
        pltpu.sync_copy(b_hbm.at[sl], zbuf)
        pltpu.sync_copy(zbuf, comb_out.at[5].at[sl])
    plsc.subcore_barrier()

    tile = cid * NTILES + sid
    sidx = (sidx0, sidx1)
    didx = (didx0, didx1)
    vals = (vals0, vals1)
    gsem = (gsem0, gsem1)
    ssem = (ssem0, ssem1)
    tsem = (tsem0, tsem1)
    _pipe(srcE, dstE, x_sp, acc_e, x_hbm, sidx, didx, vals, gsem, ssem, tsem,
          tile * EPT, EPT // CH_A, NSUB_A)
    _pipe(srcV, dstV, x_sp, acc_v, x_hbm, sidx, didx, vals, gsem, ssem, tsem,
          tile * EVPT, EVPT // CH_A, NSUB_A)
    plsc.subcore_barrier()

    pltpu.sync_copy(acc_e.at[sl], zbuf)
    pltpu.sync_copy(zbuf, comb_out.at[cid].at[sl])
    pltpu.sync_copy(acc_v.at[sl], zbuf)
    pltpu.sync_copy(zbuf, comb_out.at[cid + 2].at[sl])


def _row_segsum_body(hd_hbm, hu_hbm, srcE, dstE, srcV, dstV, accP_out,
                     sidx0, sidx1, didx0, didx1, rows0, rows1, zrow,
                     acc, gsem0, gsem1, ssem0, ssem1, tsem0, tsem1):
    cid = lax.axis_index("c")
    sid = lax.axis_index("s")

    def _zero(i, _):
        zrow[i, :] = jnp.zeros((16,), jnp.float32)
        return 0
    lax.fori_loop(0, ZR, _zero, 0)
    for k in range(NPT // ZR):
        pltpu.sync_copy(
            zrow, acc.at[pl.ds(pl.multiple_of(sid * NPT + k * ZR, ZR), ZR)])
    plsc.subcore_barrier()

    sidx = (sidx0, sidx1)
    didx = (didx0, didx1)
    rows = (rows0, rows1)
    gsem = (gsem0, gsem1)
    ssem = (ssem0, ssem1)
    tsem = (tsem0, tsem1)
    _pipe(srcE, dstE, hd_hbm, acc, hd_hbm, sidx, didx, rows, gsem, ssem, tsem,
          cid * (E_PAD // 2) + sid * EPT, EPT // CH_B, NSUB_B)
    _pipe(srcV, dstV, hu_hbm, acc, hu_hbm, sidx, didx, rows, gsem, ssem, tsem,
          cid * (EV_PAD // 2) + sid * EVPT, EVPT // CH_B, NSUB_B)
    plsc.subcore_barrier()

    for k in range(NPT // ZR):
        slk = pl.ds(pl.multiple_of(sid * NPT + k * ZR, ZR), ZR)
        pltpu.sync_copy(acc.at[slk], zrow)
        pltpu.sync_copy(zrow, accP_out.at[cid].at[slk])


@functools.cache
def _sc_kernels():
    mesh = plsc.VectorSubcoreMesh(core_axis_name="c", subcore_axis_name="s")
    sc_params = pltpu.CompilerParams(use_tc_tiling_on_sc=False)
    scalar_segsum = pl.kernel(
        _scalar_segsum_body,
        out_type=jax.ShapeDtypeStruct((8, N_PAD), jnp.float32),
        mesh=mesh,
        scratch_types=[
            pltpu.VMEM((NSUB_A, SUB), jnp.int32),
            pltpu.VMEM((NSUB_A, SUB), jnp.int32),
            pltpu.VMEM((NSUB_A, SUB), jnp.int32),
            pltpu.VMEM((NSUB_A, SUB), jnp.int32),
            pltpu.VMEM((NSUB_A, SUB), jnp.float32),
            pltpu.VMEM((NSUB_A, SUB), jnp.float32),
            pltpu.VMEM((NPT,), jnp.float32),
            pltpu.VMEM_SHARED((N_PAD,), jnp.float32),
            pltpu.VMEM_SHARED((N_PAD,), jnp.float32),
            pltpu.VMEM_SHARED((N_PAD,), jnp.float32),
            pltpu.SemaphoreType.DMA,
            pltpu.SemaphoreType.DMA,
            pltpu.SemaphoreType.DMA,
            pltpu.SemaphoreType.DMA,
            pltpu.SemaphoreType.DMA,
            pltpu.SemaphoreType.DMA,
        ],
        compiler_params=sc_params,
    )
    row_segsum = pl.kernel(
        _row_segsum_body,
        out_type=jax.ShapeDtypeStruct((NCORES, N_PAD, H), jnp.float32),
        mesh=mesh,
        scratch_types=[
            pltpu.VMEM((NSUB_B, SUB), jnp.int32),
            pltpu.VMEM((NSUB_B, SUB), jnp.int32),
            pltpu.VMEM((NSUB_B, SUB), jnp.int32),
            pltpu.VMEM((NSUB_B, SUB), jnp.int32),
            pltpu.VMEM((NSUB_B, SUB, H), jnp.float32),
            pltpu.VMEM((NSUB_B, SUB, H), jnp.float32),
            pltpu.VMEM((ZR, H), jnp.float32),
            pltpu.VMEM_SHARED((N_PAD, H), jnp.float32),
            pltpu.SemaphoreType.DMA,
            pltpu.SemaphoreType.DMA,
            pltpu.SemaphoreType.DMA,
            pltpu.SemaphoreType.DMA,
            pltpu.SemaphoreType.DMA,
            pltpu.SemaphoreType.DMA,
        ],
        compiler_params=sc_params,
    )
    return scalar_segsum, row_segsum


NB = 16
RB = N_PAD // NB


def _h_body(comb_ref, l1t_ref, wd_ref, wu_ref, ht_ref, hd_ref, hu_ref):
    se = comb_ref[0:1, :] + comb_ref[1:2, :]
    sv = comb_ref[2:3, :] + comb_ref[3:4, :]
    xr = comb_ref[4:5, :]
    p = l1t_ref[:, 0:1]
    q = l1t_ref[:, 1:2]
    r = l1t_ref[:, 2:3]
    c = l1t_ref[:, 3:4]
    ht = jnp.maximum(p * se + q * sv + r * xr + c, 0.0)
    ht_ref[...] = ht
    hd_ref[...] = jax.lax.dot_general(ht, wd_ref[...], (((0,), (0,)), ((), ())))
    hu_ref[...] = jax.lax.dot_general(ht, wu_ref[...], (((0,), (0,)), ((), ())))


def _dense_h(comb, l1t, wd, wu):
    w16 = pl.BlockSpec((H, H), lambda i: (0, 0))
    rowspec = pl.BlockSpec((RB, H), lambda i: (i, 0))
    return pl.pallas_call(
        _h_body,
        grid=(NB,),
        in_specs=[pl.BlockSpec((8, RB), lambda i: (0, i)),
                  pl.BlockSpec((H, 8), lambda i: (0, 0)), w16, w16],
        out_specs=[pl.BlockSpec((H, RB), lambda i: (0, i)), rowspec, rowspec],
        out_shape=[jax.ShapeDtypeStruct((H, N_PAD), jnp.float32),
                   jax.ShapeDtypeStruct((N_PAD, H), jnp.float32),
                   jax.ShapeDtypeStruct((N_PAD, H), jnp.float32)],
    )(comb, l1t, wd, wu)


def _final_body(ht_ref, a0_ref, a1_ref, comb_ref, wr_ref,
                b2_ref, wc_ref, bc_ref, out_ref, acc):
    i = pl.program_id(0)

    @pl.when(i == 0)
    def _():
        acc[...] = jnp.zeros((G, H), jnp.float32)

    h2 = jnp.maximum(
        a0_ref[0] + a1_ref[0]
        + jax.lax.dot_general(ht_ref[...], wr_ref[...], (((0,), (0,)), ((), ())))
        + b2_ref[0:1, :], 0.0)
    bi = lax.bitcast_convert_type(comb_ref[5:6, :], jnp.int32)
    iot = lax.broadcasted_iota(jnp.int32, (G, 1), 0)
    onehot = (bi == iot).astype(jnp.float32)
    acc[...] += jax.lax.dot_general(onehot, h2, (((1,), (0,)), ((), ())))

    @pl.when(i == NB - 1)
    def _():
        out_ref[...] = (
            jax.lax.dot_general(acc[...], wc_ref[...], (((1,), (0,)), ((), ())))
            + bc_ref[0:1, :])


def _dense_final(ht, accP, comb, wr, b2, wc_pad, bc_pad):
    return pl.pallas_call(
        _final_body,
        grid=(NB,),
        in_specs=[pl.BlockSpec((H, RB), lambda i: (0, i)),
                  pl.BlockSpec((1, RB, H), lambda i: (0, i, 0)),
                  pl.BlockSpec((1, RB, H), lambda i: (1, i, 0)),
                  pl.BlockSpec((8, RB), lambda i: (0, i)),
                  pl.BlockSpec((H, H), lambda i: (0, 0)),
                  pl.BlockSpec((8, H), lambda i: (0, 0)),
                  pl.BlockSpec((H, 128), lambda i: (0, 0)),
                  pl.BlockSpec((8, 128), lambda i: (0, 0))],
        out_specs=pl.BlockSpec((G, 128), lambda i: (0, 0)),
        out_shape=jax.ShapeDtypeStruct((G, 128), jnp.float32),
        scratch_shapes=[pltpu.VMEM((G, H), jnp.float32)],
    )(ht, accP, accP, comb, wr, b2, wc_pad, bc_pad)


def kernel(x, edge_index, v_edge_index, batch,
           W_rel_d1, b_rel_d1, W_root_d1,
           W_rel_u1, b_rel_u1, W_root_u1,
           W_rel_d2, b_rel_d2, W_root_d2,
           W_rel_u2, b_rel_u2, W_root_u2,
           W_cls, b_cls):
    x_pad = jnp.pad(x[:, 0], (0, N_PAD - N))
    batch_bits = lax.bitcast_convert_type(
        jnp.pad(batch, (0, N_PAD - N), constant_values=G), jnp.float32)
    srcE = jnp.concatenate([edge_index[0], jnp.asarray(_SRC_E_PAD)])
    dstE = jnp.concatenate([edge_index[1], jnp.asarray(_DST_E_PAD)])
    srcV = jnp.concatenate([v_edge_index[0], jnp.asarray(_SRC_V_PAD)])
    dstV = jnp.concatenate([v_edge_index[1], jnp.asarray(_DST_V_PAD)])

    scalar_segsum, row_segsum = _sc_kernels()
    comb = scalar_segsum(x_pad, batch_bits, srcE, dstE, srcV, dstV)

    l1t = jnp.zeros((H, 8), jnp.float32)
    l1t = l1t.at[:, 0].set(W_rel_d1[0]).at[:, 1].set(W_rel_u1[0])
    l1t = l1t.at[:, 2].set(W_root_d1[0] + W_root_u1[0])
    l1t = l1t.at[:, 3].set(b_rel_d1 + b_rel_u1)

    ht, hd, hu = _dense_h(comb, l1t, W_rel_d2, W_rel_u2)

    accP = row_segsum(hd, hu, srcE, dstE, srcV, dstV)

    b2 = jnp.zeros((8, H), jnp.float32).at[0].set(b_rel_d2 + b_rel_u2)
    wc_pad = jnp.zeros((H, 128), jnp.float32).at[:, :C].set(W_cls)
    bc_pad = jnp.zeros((8, 128), jnp.float32).at[0, :C].set(b_cls)
    out = _dense_final(ht, accP, comb, W_root_d2 + W_root_u2,
                       b2, wc_pad, bc_pad)
    return out[:, :C]


# speedup vs baseline: 117.1896x; 1.0638x over previous
import functools

import jax
import jax.numpy as jnp
import numpy as np
from jax import lax
from jax.experimental import pallas as pl
from jax.experimental.pallas import tpu as pltpu
from jax.experimental.pallas import tpu_sc as plsc

N = 50000
E = 3200000
EV = 1600000
G = 128
H = 16
C = 2

NCORES = 2
NTILES = 16
NPT = 3200
N_PAD = NTILES * NPT
SUB = 128
ZR = 400

CH_A = 2048
NSUB_A = CH_A // SUB
CH_B = 1024
NSUB_B = CH_B // SUB

E_ROWS = E // SUB
EV_ROWS = EV // SUB
E_MAIN = 768
V_MAIN = 384
E_TAIL, E_TAILW = 53, 8
V_TAIL, V_TAILW = 53, 4
NLAST = N - 15 * NPT


def _pipe(src_hbm, dst_hbm, gtab, acc, dummy_hbm,
          sidx, didx, vals, gsem, ssem, tsem, edge_off, nchunks, nsub):
    ch = nsub * SUB
    nch2 = nchunks // 2

    def stage(g, b):
        for j in range(nsub):
            e0 = pl.multiple_of(edge_off + g * ch + j * SUB, SUB)
            pltpu.async_copy(src_hbm.at[pl.ds(e0, SUB)], sidx[b].at[j], tsem[b])
            pltpu.async_copy(dst_hbm.at[pl.ds(e0, SUB)], didx[b].at[j], tsem[b])

    def drain_stage(b):
        for j in range(2 * nsub):
            pltpu.make_async_copy(
                src_hbm.at[pl.ds(0, SUB)], sidx[b].at[0], tsem[b]).wait()

    def fire_gathers(b):
        for j in range(nsub):
            pltpu.async_copy(gtab.at[sidx[b].at[j]], vals[b].at[j], gsem[b])

    def fire_scatters(b):
        for j in range(nsub):
            pltpu.async_copy(vals[b].at[j], acc.at[didx[b].at[j]],
                             ssem[b], add=True)

    def drain(sem, b):
        for j in range(nsub):
            pltpu.make_async_copy(
                dummy_hbm.at[pl.ds(0, SUB)], vals[b].at[j], sem).wait()

    stage(0, 0)
    drain_stage(0)
    fire_gathers(0)

    def pair(i, _):
        @pl.when(i > 0)
        def _():
            drain(ssem[1], 1)
        stage(2 * i + 1, 1)
        drain(gsem[0], 0)
        fire_scatters(0)
        drain_stage(1)
        fire_gathers(1)

        @pl.when(i < nch2 - 1)
        def _():
            drain(ssem[0], 0)
            stage(2 * i + 2, 0)
        drain(gsem[1], 1)
        fire_scatters(1)

        @pl.when(i < nch2 - 1)
        def _():
            drain_stage(0)
            fire_gathers(0)
        return 0

    lax.fori_loop(0, nch2, pair, 0)
    drain(ssem[0], 0)
    drain(ssem[1], 1)


def _tail(src_hbm, dst_hbm, gtab, acc, dummy_hbm,
          sidx, didx, vals, gsem, ssem, tsem, row_base, nrows, nsub):
    done = 0
    while done < nrows:
        m = min(nsub, nrows - done)
        for j in range(m):
            e0 = pl.multiple_of((row_base + done + j) * SUB, SUB)
            pltpu.async_copy(src_hbm.at[pl.ds(e0, SUB)], sidx[0].at[j], tsem[0])
            pltpu.async_copy(dst_hbm.at[pl.ds(e0, SUB)], didx[0].at[j], tsem[0])
        for j in range(2 * m):
            pltpu.make_async_copy(
                src_hbm.at[pl.ds(0, SUB)], sidx[0].at[0], tsem[0]).wait()
        for j in range(m):
            pltpu.async_copy(gtab.at[sidx[0].at[j]], vals[0].at[j], gsem[0])
        for j in range(m):
            pltpu.make_async_copy(
                dummy_hbm.at[pl.ds(0, SUB)], vals[0].at[j], gsem[0]).wait()
        for j in range(m):
            pltpu.async_copy(vals[0].at[j], acc.at[didx[0].at[j]],
                             ssem[0], add=True)
        for j in range(m):
            pltpu.make_async_copy(
                dummy_hbm.at[pl.ds(0, SUB)], vals[0].at[j], ssem[0]).wait()
        done += m


def _scalar_segsum_body(x_hbm, b_hbm, eiE, eiV, comb_out,
                        sidx0, sidx1, didx0, didx1, vals0, vals1, zbuf,
                        x_sp, acc_e, acc_v, gsem0, gsem1, ssem0, ssem1,
                        tsem0, tsem1):
    cid = lax.axis_index("c")
    sid = lax.axis_index("s")

    def _zero(i, _):
        zbuf[pl.ds(pl.multiple_of(i * 16, 16), 16)] = jnp.zeros((16,), jnp.float32)
        return 0
    lax.fori_loop(0, NPT // 16, _zero, 0)
    sl = pl.ds(pl.multiple_of(sid * NPT, NPT), NPT)
    pltpu.sync_copy(zbuf, acc_e.at[sl])
    pltpu.sync_copy(zbuf, acc_v.at[sl])
    @pl.when(sid < NTILES - 1)
    def _():
        pltpu.sync_copy(x_hbm.at[sl], zbuf)

    @pl.when(sid == NTILES - 1)
    def _():
        pltpu.sync_copy(x_hbm.at[pl.ds((NTILES - 1) * NPT, NLAST)],
                        zbuf.at[pl.ds(0, NLAST)])
    pltpu.sync_copy(zbuf, x_sp.at[sl])

    @pl.when(cid == 0)
    def _():
        pltpu.sync_copy(zbuf, comb_out.at[4].at[sl])

        @pl.when(sid < NTILES - 1)
        def _():
            pltpu.sync_copy(b_hbm.at[sl], zbuf)

        @pl.when(sid == NTILES - 1)
        def _():
            pltpu.sync_copy(b_hbm.at[pl.ds((NTILES - 1) * NPT, NLAST)],
                            zbuf.at[pl.ds(0, NLAST)])
        pltpu.sync_copy(zbuf, comb_out.at[5].at[sl])
    plsc.subcore_barrier()

    w = cid * NTILES + sid
    sidx = (sidx0, sidx1)
    didx = (didx0, didx1)
    vals = (vals0, vals1)
    gsem = (gsem0, gsem1)
    ssem = (ssem0, ssem1)
    tsem = (tsem0, tsem1)
    args_e = (eiE.at[0], eiE.at[1], x_sp, acc_e, x_hbm,
              sidx, didx, vals, gsem, ssem, tsem)
    args_v = (eiV.at[0], eiV.at[1], x_sp, acc_v, x_hbm,
              sidx, didx, vals, gsem, ssem, tsem)
    _pipe(*args_e, w * E_MAIN * SUB, E_MAIN // NSUB_A, NSUB_A)
    _pipe(*args_v, w * V_MAIN * SUB, V_MAIN // NSUB_A, NSUB_A)

    @pl.when(w < E_TAILW)
    def _():
        _tail(*args_e, 32 * E_MAIN + w * E_TAIL, E_TAIL, NSUB_A)

    @pl.when(w < V_TAILW)
    def _():
        _tail(*args_v, 32 * V_MAIN + w * V_TAIL, V_TAIL, NSUB_A)
    plsc.subcore_barrier()

    pltpu.sync_copy(acc_e.at[sl], zbuf)
    pltpu.sync_copy(zbuf, comb_out.at[cid].at[sl])
    pltpu.sync_copy(acc_v.at[sl], zbuf)
    pltpu.sync_copy(zbuf, comb_out.at[cid + 2].at[sl])


def _row_segsum_body(hd_hbm, hu_hbm, eiE, eiV, accP_out,
                     sidx0, sidx1, didx0, didx1, rows0, rows1, zrow,
                     acc, gsem0, gsem1, ssem0, ssem1, tsem0, tsem1):
    cid = lax.axis_index("c")
    sid = lax.axis_index("s")

    def _zero(i, _):
        zrow[i, :] = jnp.zeros((16,), jnp.float32)
        return 0
    lax.fori_loop(0, ZR, _zero, 0)
    for k in range(NPT // ZR):
        pltpu.sync_copy(
            zrow, acc.at[pl.ds(pl.multiple_of(sid * NPT + k * ZR, ZR), ZR)])
    plsc.subcore_barrier()

    w = cid * NTILES + sid
    sidx = (sidx0, sidx1)
    didx = (didx0, didx1)
    rows = (rows0, rows1)
    gsem = (gsem0, gsem1)
    ssem = (ssem0, ssem1)
    tsem = (tsem0, tsem1)
    args_e = (eiE.at[0], eiE.at[1], hd_hbm, acc, hd_hbm,
              sidx, didx, rows, gsem, ssem, tsem)
    args_v = (eiV.at[0], eiV.at[1], hu_hbm, acc, hu_hbm,
              sidx, didx, rows, gsem, ssem, tsem)
    _pipe(*args_e, w * E_MAIN * SUB, E_MAIN // NSUB_B, NSUB_B)
    _pipe(*args_v, w * V_MAIN * SUB, V_MAIN // NSUB_B, NSUB_B)

    @pl.when(w < E_TAILW)
    def _():
        _tail(*args_e, 32 * E_MAIN + w * E_TAIL, E_TAIL, NSUB_B)

    @pl.when(w < V_TAILW)
    def _():
        _tail(*args_v, 32 * V_MAIN + w * V_TAIL, V_TAIL, NSUB_B)
    plsc.subcore_barrier()

    for k in range(NPT // ZR):
        slk = pl.ds(pl.multiple_of(sid * NPT + k * ZR, ZR), ZR)
        pltpu.sync_copy(acc.at[slk], zrow)
        pltpu.sync_copy(zrow, accP_out.at[cid].at[slk])


@functools.cache
def _sc_kernels():
    mesh = plsc.VectorSubcoreMesh(core_axis_name="c", subcore_axis_name="s")
    sc_params = pltpu.CompilerParams(use_tc_tiling_on_sc=False)
    scalar_segsum = pl.kernel(
        _scalar_segsum_body,
        out_type=jax.ShapeDtypeStruct((8, N_PAD), jnp.float32),
        mesh=mesh,
        scratch_types=[
            pltpu.VMEM((NSUB_A, SUB), jnp.int32),
            pltpu.VMEM((NSUB_A, SUB), jnp.int32),
            pltpu.VMEM((NSUB_A, SUB), jnp.int32),
            pltpu.VMEM((NSUB_A, SUB), jnp.int32),
            pltpu.VMEM((NSUB_A, SUB), jnp.float32),
            pltpu.VMEM((NSUB_A, SUB), jnp.float32),
            pltpu.VMEM((NPT,), jnp.float32),
            pltpu.VMEM_SHARED((N_PAD,), jnp.float32),
            pltpu.VMEM_SHARED((N_PAD,), jnp.float32),
            pltpu.VMEM_SHARED((N_PAD,), jnp.float32),
            pltpu.SemaphoreType.DMA,
            pltpu.SemaphoreType.DMA,
            pltpu.SemaphoreType.DMA,
            pltpu.SemaphoreType.DMA,
            pltpu.SemaphoreType.DMA,
            pltpu.SemaphoreType.DMA,
        ],
        compiler_params=sc_params,
    )
    row_segsum = pl.kernel(
        _row_segsum_body,
        out_type=jax.ShapeDtypeStruct((NCORES, N_PAD, H), jnp.float32),
        mesh=mesh,
        scratch_types=[
            pltpu.VMEM((NSUB_B, SUB), jnp.int32),
            pltpu.VMEM((NSUB_B, SUB), jnp.int32),
            pltpu.VMEM((NSUB_B, SUB), jnp.int32),
            pltpu.VMEM((NSUB_B, SUB), jnp.int32),
            pltpu.VMEM((NSUB_B, SUB, H), jnp.float32),
            pltpu.VMEM((NSUB_B, SUB, H), jnp.float32),
            pltpu.VMEM((ZR, H), jnp.float32),
            pltpu.VMEM_SHARED((N_PAD, H), jnp.float32),
            pltpu.SemaphoreType.DMA,
            pltpu.SemaphoreType.DMA,
            pltpu.SemaphoreType.DMA,
            pltpu.SemaphoreType.DMA,
            pltpu.SemaphoreType.DMA,
            pltpu.SemaphoreType.DMA,
        ],
        compiler_params=sc_params,
    )
    return scalar_segsum, row_segsum


NB = 16
RB = N_PAD // NB


def _h_body(comb_ref, l1t_ref, wd_ref, wu_ref, ht_ref, hd_ref, hu_ref):
    se = comb_ref[0:1, :] + comb_ref[1:2, :]
    sv = comb_ref[2:3, :] + comb_ref[3:4, :]
    xr = comb_ref[4:5, :]
    p = l1t_ref[:, 0:1]
    q = l1t_ref[:, 1:2]
    r = l1t_ref[:, 2:3]
    c = l1t_ref[:, 3:4]
    ht = jnp.maximum(p * se + q * sv + r * xr + c, 0.0)
    ht_ref[...] = ht
    hd_ref[...] = jax.lax.dot_general(ht, wd_ref[...], (((0,), (0,)), ((), ())))
    hu_ref[...] = jax.lax.dot_general(ht, wu_ref[...], (((0,), (0,)), ((), ())))


def _dense_h(comb, l1t, wd, wu):
    w16 = pl.BlockSpec((H, H), lambda i: (0, 0))
    rowspec = pl.BlockSpec((RB, H), lambda i: (i, 0))
    return pl.pallas_call(
        _h_body,
        grid=(NB,),
        in_specs=[pl.BlockSpec((8, RB), lambda i: (0, i)),
                  pl.BlockSpec((H, 8), lambda i: (0, 0)), w16, w16],
        out_specs=[pl.BlockSpec((H, RB), lambda i: (0, i)), rowspec, rowspec],
        out_shape=[jax.ShapeDtypeStruct((H, N_PAD), jnp.float32),
                   jax.ShapeDtypeStruct((N_PAD, H), jnp.float32),
                   jax.ShapeDtypeStruct((N_PAD, H), jnp.float32)],
    )(comb, l1t, wd, wu)


def _final_body(ht_ref, a0_ref, a1_ref, comb_ref, wr_ref,
                b2_ref, wc_ref, bc_ref, out_ref, acc):
    i = pl.program_id(0)

    @pl.when(i == 0)
    def _():
        acc[...] = jnp.zeros((G, H), jnp.float32)

    h2 = jnp.maximum(
        a0_ref[0] + a1_ref[0]
        + jax.lax.dot_general(ht_ref[...], wr_ref[...], (((0,), (0,)), ((), ())))
        + b2_ref[0:1, :], 0.0)
    bi = lax.bitcast_convert_type(comb_ref[5:6, :], jnp.int32)
    iot = lax.broadcasted_iota(jnp.int32, (G, 1), 0)
    rid = lax.broadcasted_iota(jnp.int32, (1, RB), 1) + i * RB
    onehot = ((bi == iot) & (rid < N)).astype(jnp.float32)
    acc[...] += jax.lax.dot_general(onehot, h2, (((1,), (0,)), ((), ())))

    @pl.when(i == NB - 1)
    def _():
        out_ref[...] = (
            jax.lax.dot_general(acc[...], wc_ref[...], (((1,), (0,)), ((), ())))
            + bc_ref[0:1, :])


def _dense_final(ht, accP, comb, wr, b2, wc_pad, bc_pad):
    return pl.pallas_call(
        _final_body,
        grid=(NB,),
        in_specs=[pl.BlockSpec((H, RB), lambda i: (0, i)),
                  pl.BlockSpec((1, RB, H), lambda i: (0, i, 0)),
                  pl.BlockSpec((1, RB, H), lambda i: (1, i, 0)),
                  pl.BlockSpec((8, RB), lambda i: (0, i)),
                  pl.BlockSpec((H, H), lambda i: (0, 0)),
                  pl.BlockSpec((8, H), lambda i: (0, 0)),
                  pl.BlockSpec((H, 128), lambda i: (0, 0)),
                  pl.BlockSpec((8, 128), lambda i: (0, 0))],
        out_specs=pl.BlockSpec((G, 128), lambda i: (0, 0)),
        out_shape=jax.ShapeDtypeStruct((G, 128), jnp.float32),
        scratch_shapes=[pltpu.VMEM((G, H), jnp.float32)],
    )(ht, accP, accP, comb, wr, b2, wc_pad, bc_pad)


def kernel(x, edge_index, v_edge_index, batch,
           W_rel_d1, b_rel_d1, W_root_d1,
           W_rel_u1, b_rel_u1, W_root_u1,
           W_rel_d2, b_rel_d2, W_root_d2,
           W_rel_u2, b_rel_u2, W_root_u2,
           W_cls, b_cls):
    xf = x[:, 0]
    batch_bits = lax.bitcast_convert_type(batch, jnp.float32)

    scalar_segsum, row_segsum = _sc_kernels()
    comb = scalar_segsum(xf, batch_bits, edge_index, v_edge_index)

    l1t = jnp.zeros((H, 8), jnp.float32)
    l1t = l1t.at[:, 0].set(W_rel_d1[0]).at[:, 1].set(W_rel_u1[0])
    l1t = l1t.at[:, 2].set(W_root_d1[0] + W_root_u1[0])
    l1t = l1t.at[:, 3].set(b_rel_d1 + b_rel_u1)

    ht, hd, hu = _dense_h(comb, l1t, W_rel_d2, W_rel_u2)

    accP = row_segsum(hd, hu, edge_index, v_edge_index)

    b2 = jnp.zeros((8, H), jnp.float32).at[0].set(b_rel_d2 + b_rel_u2)
    wc_pad = jnp.zeros((H, 128), jnp.float32).at[:, :C].set(W_cls)
    bc_pad = jnp.zeros((8, 128), jnp.float32).at[0, :C].set(b_cls)
    out = _dense_final(ht, accP, comb, W_root_d2 + W_root_u2,
                       b2, wc_pad, bc_pad)
    return out[:, :C]


# speedup vs baseline: 126.1171x; 1.0762x over previous
import functools

import jax
import jax.numpy as jnp
import numpy as np
from jax import lax
from jax.experimental import pallas as pl
from jax.experimental.pallas import tpu as pltpu
from jax.experimental.pallas import tpu_sc as plsc

N = 50000
E = 3200000
EV = 1600000
G = 128
H = 16
C = 2

NCORES = 2
NTILES = 16
NPT = 3200
N_PAD = NTILES * NPT
SUB = 128
ZR = 400

CH_A = 2048
NSUB_A = CH_A // SUB
CH_B = 1024
NSUB_B = CH_B // SUB

E_ROWS = E // SUB
EV_ROWS = EV // SUB
E_MAIN = 768
V_MAIN = 384
NLAST = N - 15 * NPT


def _pipe(src_hbm, dst_hbm, gtab, acc, dummy_hbm,
          sidx, didx, vals, gsem, ssem, tsem, edge_off, nchunks, nsub):
    ch = nsub * SUB
    nch2 = nchunks // 2

    def stage(g, b):
        for j in range(nsub):
            e0 = pl.multiple_of(edge_off + g * ch + j * SUB, SUB)
            pltpu.async_copy(src_hbm.at[pl.ds(e0, SUB)], sidx[b].at[j], tsem[b])
            pltpu.async_copy(dst_hbm.at[pl.ds(e0, SUB)], didx[b].at[j], tsem[b])

    def drain_stage(b):
        for j in range(2 * nsub):
            pltpu.make_async_copy(
                src_hbm.at[pl.ds(0, SUB)], sidx[b].at[0], tsem[b]).wait()

    def fire_gathers(b):
        for j in range(nsub):
            pltpu.async_copy(gtab.at[sidx[b].at[j]], vals[b].at[j], gsem[b])

    def fire_scatters(b):
        for j in range(nsub):
            pltpu.async_copy(vals[b].at[j], acc.at[didx[b].at[j]],
                             ssem[b], add=True)

    def drain(sem, b):
        for j in range(nsub):
            pltpu.make_async_copy(
                dummy_hbm.at[pl.ds(0, SUB)], vals[b].at[j], sem).wait()

    stage(0, 0)
    drain_stage(0)
    fire_gathers(0)

    def pair(i, _):
        @pl.when(i > 0)
        def _():
            drain(ssem[1], 1)
        stage(2 * i + 1, 1)
        drain(gsem[0], 0)
        fire_scatters(0)
        drain_stage(1)
        fire_gathers(1)

        @pl.when(i < nch2 - 1)
        def _():
            drain(ssem[0], 0)
            stage(2 * i + 2, 0)
        drain(gsem[1], 1)
        fire_scatters(1)

        @pl.when(i < nch2 - 1)
        def _():
            drain_stage(0)
            fire_gathers(0)
        return 0

    lax.fori_loop(0, nch2, pair, 0)
    drain(ssem[0], 0)
    drain(ssem[1], 1)


def _tail(src_hbm, dst_hbm, gtab, acc, dummy_hbm,
          sidx, didx, vals, gsem, ssem, tsem, row_base, nrows, nsub):
    done = 0
    while done < nrows:
        m = min(nsub, nrows - done)
        for j in range(m):
            e0 = pl.multiple_of((row_base + done + j) * SUB, SUB)
            pltpu.async_copy(src_hbm.at[pl.ds(e0, SUB)], sidx[0].at[j], tsem[0])
            pltpu.async_copy(dst_hbm.at[pl.ds(e0, SUB)], didx[0].at[j], tsem[0])
        for j in range(2 * m):
            pltpu.make_async_copy(
                src_hbm.at[pl.ds(0, SUB)], sidx[0].at[0], tsem[0]).wait()
        for j in range(m):
            pltpu.async_copy(gtab.at[sidx[0].at[j]], vals[0].at[j], gsem[0])
        for j in range(m):
            pltpu.make_async_copy(
                dummy_hbm.at[pl.ds(0, SUB)], vals[0].at[j], gsem[0]).wait()
        for j in range(m):
            pltpu.async_copy(vals[0].at[j], acc.at[didx[0].at[j]],
                             ssem[0], add=True)
        for j in range(m):
            pltpu.make_async_copy(
                dummy_hbm.at[pl.ds(0, SUB)], vals[0].at[j], ssem[0]).wait()
        done += m


def _tails(args_e, args_v, w, nsub):
    _tail(*args_e, 32 * E_MAIN + 13 * w, 13, nsub)

    @pl.when(w < 8)
    def _():
        _tail(*args_e, 32 * E_MAIN + 416 + w, 1, nsub)
    _tail(*args_v, 32 * V_MAIN + 6 * w, 6, nsub)

    @pl.when(w < 20)
    def _():
        _tail(*args_v, 32 * V_MAIN + 192 + w, 1, nsub)


def _scalar_segsum_body(x_hbm, b_hbm, eiE, eiV, comb_out,
                        sidx0, sidx1, didx0, didx1, vals0, vals1, zbuf,
                        x_sp, acc_e, acc_v, gsem0, gsem1, ssem0, ssem1,
                        tsem0, tsem1):
    cid = lax.axis_index("c")
    sid = lax.axis_index("s")

    def _zero(i, _):
        zbuf[pl.ds(pl.multiple_of(i * 16, 16), 16)] = jnp.zeros((16,), jnp.float32)
        return 0
    lax.fori_loop(0, NPT // 16, _zero, 0)
    sl = pl.ds(pl.multiple_of(sid * NPT, NPT), NPT)
    pltpu.sync_copy(zbuf, acc_e.at[sl])
    pltpu.sync_copy(zbuf, acc_v.at[sl])
    @pl.when(sid < NTILES - 1)
    def _():
        pltpu.sync_copy(x_hbm.at[sl], zbuf)

    @pl.when(sid == NTILES - 1)
    def _():
        pltpu.sync_copy(x_hbm.at[pl.ds((NTILES - 1) * NPT, NLAST)],
                        zbuf.at[pl.ds(0, NLAST)])
    pltpu.sync_copy(zbuf, x_sp.at[sl])

    @pl.when(cid == 0)
    def _():
        pltpu.sync_copy(zbuf, comb_out.at[4].at[sl])

        @pl.when(sid < NTILES - 1)
        def _():
            pltpu.sync_copy(b_hbm.at[sl], zbuf)

        @pl.when(sid == NTILES - 1)
        def _():
            pltpu.sync_copy(b_hbm.at[pl.ds((NTILES - 1) * NPT, NLAST)],
                            zbuf.at[pl.ds(0, NLAST)])
        pltpu.sync_copy(zbuf, comb_out.at[5].at[sl])
    plsc.subcore_barrier()

    w = cid * NTILES + sid
    sidx = (sidx0, sidx1)
    didx = (didx0, didx1)
    vals = (vals0, vals1)
    gsem = (gsem0, gsem1)
    ssem = (ssem0, ssem1)
    tsem = (tsem0, tsem1)
    args_e = (eiE.at[0], eiE.at[1], x_sp, acc_e, x_hbm,
              sidx, didx, vals, gsem, ssem, tsem)
    args_v = (eiV.at[0], eiV.at[1], x_sp, acc_v, x_hbm,
              sidx, didx, vals, gsem, ssem, tsem)
    _pipe(*args_e, w * E_MAIN * SUB, E_MAIN // NSUB_A, NSUB_A)
    _pipe(*args_v, w * V_MAIN * SUB, V_MAIN // NSUB_A, NSUB_A)
    _tails(args_e, args_v, w, NSUB_A)
    plsc.subcore_barrier()

    pltpu.sync_copy(acc_e.at[sl], zbuf)
    pltpu.sync_copy(zbuf, comb_out.at[cid].at[sl])
    pltpu.sync_copy(acc_v.at[sl], zbuf)
    pltpu.sync_copy(zbuf, comb_out.at[cid + 2].at[sl])


def _row_segsum_body(hd_hbm, hu_hbm, eiE, eiV, accP_out,
                     sidx0, sidx1, didx0, didx1, rows0, rows1, zrow,
                     acc, gsem0, gsem1, ssem0, ssem1, tsem0, tsem1):
    cid = lax.axis_index("c")
    sid = lax.axis_index("s")

    def _zero(i, _):
        zrow[i, :] = jnp.zeros((16,), jnp.float32)
        return 0
    lax.fori_loop(0, ZR, _zero, 0)
    for k in range(NPT // ZR):
        pltpu.sync_copy(
            zrow, acc.at[pl.ds(pl.multiple_of(sid * NPT + k * ZR, ZR), ZR)])
    plsc.subcore_barrier()

    w = cid * NTILES + sid
    sidx = (sidx0, sidx1)
    didx = (didx0, didx1)
    rows = (rows0, rows1)
    gsem = (gsem0, gsem1)
    ssem = (ssem0, ssem1)
    tsem = (tsem0, tsem1)
    args_e = (eiE.at[0], eiE.at[1], hd_hbm, acc, hd_hbm,
              sidx, didx, rows, gsem, ssem, tsem)
    args_v = (eiV.at[0], eiV.at[1], hu_hbm, acc, hu_hbm,
              sidx, didx, rows, gsem, ssem, tsem)
    _pipe(*args_e, w * E_MAIN * SUB, E_MAIN // NSUB_B, NSUB_B)
    _pipe(*args_v, w * V_MAIN * SUB, V_MAIN // NSUB_B, NSUB_B)
    _tails(args_e, args_v, w, NSUB_B)
    plsc.subcore_barrier()

    for k in range(NPT // ZR):
        slk = pl.ds(pl.multiple_of(sid * NPT + k * ZR, ZR), ZR)
        pltpu.sync_copy(acc.at[slk], zrow)
        pltpu.sync_copy(zrow, accP_out.at[cid].at[slk])


@functools.cache
def _sc_kernels():
    mesh = plsc.VectorSubcoreMesh(core_axis_name="c", subcore_axis_name="s")
    sc_params = pltpu.CompilerParams(use_tc_tiling_on_sc=False)
    scalar_segsum = pl.kernel(
        _scalar_segsum_body,
        out_type=jax.ShapeDtypeStruct((8, N_PAD), jnp.float32),
        mesh=mesh,
        scratch_types=[
            pltpu.VMEM((NSUB_A, SUB), jnp.int32),
            pltpu.VMEM((NSUB_A, SUB), jnp.int32),
            pltpu.VMEM((NSUB_A, SUB), jnp.int32),
            pltpu.VMEM((NSUB_A, SUB), jnp.int32),
            pltpu.VMEM((NSUB_A, SUB), jnp.float32),
            pltpu.VMEM((NSUB_A, SUB), jnp.float32),
            pltpu.VMEM((NPT,), jnp.float32),
            pltpu.VMEM_SHARED((N_PAD,), jnp.float32),
            pltpu.VMEM_SHARED((N_PAD,), jnp.float32),
            pltpu.VMEM_SHARED((N_PAD,), jnp.float32),
            pltpu.SemaphoreType.DMA,
            pltpu.SemaphoreType.DMA,
            pltpu.SemaphoreType.DMA,
            pltpu.SemaphoreType.DMA,
            pltpu.SemaphoreType.DMA,
            pltpu.SemaphoreType.DMA,
        ],
        compiler_params=sc_params,
    )
    row_segsum = pl.kernel(
        _row_segsum_body,
        out_type=jax.ShapeDtypeStruct((NCORES, N_PAD, H), jnp.float32),
        mesh=mesh,
        scratch_types=[
            pltpu.VMEM((NSUB_B, SUB), jnp.int32),
            pltpu.VMEM((NSUB_B, SUB), jnp.int32),
            pltpu.VMEM((NSUB_B, SUB), jnp.int32),
            pltpu.VMEM((NSUB_B, SUB), jnp.int32),
            pltpu.VMEM((NSUB_B, SUB, H), jnp.float32),
            pltpu.VMEM((NSUB_B, SUB, H), jnp.float32),
            pltpu.VMEM((ZR, H), jnp.float32),
            pltpu.VMEM_SHARED((N_PAD, H), jnp.float32),
            pltpu.SemaphoreType.DMA,
            pltpu.SemaphoreType.DMA,
            pltpu.SemaphoreType.DMA,
            pltpu.SemaphoreType.DMA,
            pltpu.SemaphoreType.DMA,
            pltpu.SemaphoreType.DMA,
        ],
        compiler_params=sc_params,
    )
    return scalar_segsum, row_segsum


NB = 16
RB = N_PAD // NB


def _h_body(comb_ref, l1t_ref, wd_ref, wu_ref, ht_ref, hd_ref, hu_ref):
    se = comb_ref[0:1, :] + comb_ref[1:2, :]
    sv = comb_ref[2:3, :] + comb_ref[3:4, :]
    xr = comb_ref[4:5, :]
    p = l1t_ref[:, 0:1]
    q = l1t_ref[:, 1:2]
    r = l1t_ref[:, 2:3]
    c = l1t_ref[:, 3:4]
    ht = jnp.maximum(p * se + q * sv + r * xr + c, 0.0)
    ht_ref[...] = ht
    hd_ref[...] = jax.lax.dot_general(ht, wd_ref[...], (((0,), (0,)), ((), ())))
    hu_ref[...] = jax.lax.dot_general(ht, wu_ref[...], (((0,), (0,)), ((), ())))


def _dense_h(comb, l1t, wd, wu):
    w16 = pl.BlockSpec((H, H), lambda i: (0, 0))
    rowspec = pl.BlockSpec((RB, H), lambda i: (i, 0))
    return pl.pallas_call(
        _h_body,
        grid=(NB,),
        in_specs=[pl.BlockSpec((8, RB), lambda i: (0, i)),
                  pl.BlockSpec((H, 8), lambda i: (0, 0)), w16, w16],
        out_specs=[pl.BlockSpec((H, RB), lambda i: (0, i)), rowspec, rowspec],
        out_shape=[jax.ShapeDtypeStruct((H, N_PAD), jnp.float32),
                   jax.ShapeDtypeStruct((N_PAD, H), jnp.float32),
                   jax.ShapeDtypeStruct((N_PAD, H), jnp.float32)],
    )(comb, l1t, wd, wu)


def _final_body(ht_ref, a0_ref, a1_ref, comb_ref, wr_ref,
                b2_ref, wc_ref, bc_ref, out_ref, acc):
    i = pl.program_id(0)

    @pl.when(i == 0)
    def _():
        acc[...] = jnp.zeros((G, H), jnp.float32)

    h2 = jnp.maximum(
        a0_ref[0] + a1_ref[0]
        + jax.lax.dot_general(ht_ref[...], wr_ref[...], (((0,), (0,)), ((), ())))
        + b2_ref[0:1, :], 0.0)
    bi = lax.bitcast_convert_type(comb_ref[5:6, :], jnp.int32)
    iot = lax.broadcasted_iota(jnp.int32, (G, 1), 0)
    rid = lax.broadcasted_iota(jnp.int32, (1, RB), 1) + i * RB
    onehot = ((bi == iot) & (rid < N)).astype(jnp.float32)
    acc[...] += jax.lax.dot_general(onehot, h2, (((1,), (0,)), ((), ())))

    @pl.when(i == NB - 1)
    def _():
        out_ref[...] = (
            jax.lax.dot_general(acc[...], wc_ref[...], (((1,), (0,)), ((), ())))
            + bc_ref[0:1, :])


def _dense_final(ht, accP, comb, wr, b2, wc_pad, bc_pad):
    return pl.pallas_call(
        _final_body,
        grid=(NB,),
        in_specs=[pl.BlockSpec((H, RB), lambda i: (0, i)),
                  pl.BlockSpec((1, RB, H), lambda i: (0, i, 0)),
                  pl.BlockSpec((1, RB, H), lambda i: (1, i, 0)),
                  pl.BlockSpec((8, RB), lambda i: (0, i)),
                  pl.BlockSpec((H, H), lambda i: (0, 0)),
                  pl.BlockSpec((8, H), lambda i: (0, 0)),
                  pl.BlockSpec((H, 128), lambda i: (0, 0)),
                  pl.BlockSpec((8, 128), lambda i: (0, 0))],
        out_specs=pl.BlockSpec((G, 128), lambda i: (0, 0)),
        out_shape=jax.ShapeDtypeStruct((G, 128), jnp.float32),
        scratch_shapes=[pltpu.VMEM((G, H), jnp.float32)],
    )(ht, accP, accP, comb, wr, b2, wc_pad, bc_pad)


def kernel(x, edge_index, v_edge_index, batch,
           W_rel_d1, b_rel_d1, W_root_d1,
           W_rel_u1, b_rel_u1, W_root_u1,
           W_rel_d2, b_rel_d2, W_root_d2,
           W_rel_u2, b_rel_u2, W_root_u2,
           W_cls, b_cls):
    xf = x[:, 0]
    batch_bits = lax.bitcast_convert_type(batch, jnp.float32)

    scalar_segsum, row_segsum = _sc_kernels()
    comb = scalar_segsum(xf, batch_bits, edge_index, v_edge_index)

    l1t = jnp.zeros((H, 8), jnp.float32)
    l1t = l1t.at[:, 0].set(W_rel_d1[0]).at[:, 1].set(W_rel_u1[0])
    l1t = l1t.at[:, 2].set(W_root_d1[0] + W_root_u1[0])
    l1t = l1t.at[:, 3].set(b_rel_d1 + b_rel_u1)

    ht, hd, hu = _dense_h(comb, l1t, W_rel_d2, W_rel_u2)

    accP = row_segsum(hd, hu, edge_index, v_edge_index)

    b2 = jnp.zeros((8, H), jnp.float32).at[0].set(b_rel_d2 + b_rel_u2)
    wc_pad = jnp.zeros((H, 128), jnp.float32).at[:, :C].set(W_cls)
    bc_pad = jnp.zeros((8, 128), jnp.float32).at[0, :C].set(b_cls)
    out = _dense_final(ht, accP, comb, W_root_d2 + W_root_u2,
                       b2, wc_pad, bc_pad)
    return out[:, :C]


# speedup vs baseline: 126.3303x; 1.0017x over previous
import functools

import jax
import jax.numpy as jnp
import numpy as np
from jax import lax
from jax.experimental import pallas as pl
from jax.experimental.pallas import tpu as pltpu
from jax.experimental.pallas import tpu_sc as plsc

N = 50000
E = 3200000
EV = 1600000
G = 128
H = 16
C = 2

NCORES = 2
NTILES = 16
NPT = 3200
N_PAD = NTILES * NPT
SUB = 128
ZR = 400

CH_A = 4096
NSUB_A = CH_A // SUB
CH_B = 1024
NSUB_B = CH_B // SUB

E_ROWS = E // SUB
EV_ROWS = EV // SUB
E_MAIN = 768
V_MAIN = 384
NLAST = N - 15 * NPT


def _pipe(src_hbm, dst_hbm, gtab, acc, dummy_hbm,
          sidx, didx, vals, gsem, ssem, tsem, edge_off, nchunks, nsub):
    ch = nsub * SUB
    nch2 = nchunks // 2

    def stage(g, b):
        for j in range(nsub):
            e0 = pl.multiple_of(edge_off + g * ch + j * SUB, SUB)
            pltpu.async_copy(src_hbm.at[pl.ds(e0, SUB)], sidx[b].at[j], tsem[b])
            pltpu.async_copy(dst_hbm.at[pl.ds(e0, SUB)], didx[b].at[j], tsem[b])

    def drain_stage(b):
        for j in range(2 * nsub):
            pltpu.make_async_copy(
                src_hbm.at[pl.ds(0, SUB)], sidx[b].at[0], tsem[b]).wait()

    def fire_gathers(b):
        for j in range(nsub):
            pltpu.async_copy(gtab.at[sidx[b].at[j]], vals[b].at[j], gsem[b])

    def fire_scatters(b):
        for j in range(nsub):
            pltpu.async_copy(vals[b].at[j], acc.at[didx[b].at[j]],
                             ssem[b], add=True)

    def drain(sem, b):
        for j in range(nsub):
            pltpu.make_async_copy(
                dummy_hbm.at[pl.ds(0, SUB)], vals[b].at[j], sem).wait()

    stage(0, 0)
    drain_stage(0)
    fire_gathers(0)

    def pair(i, _):
        @pl.when(i > 0)
        def _():
            drain(ssem[1], 1)
        stage(2 * i + 1, 1)
        drain(gsem[0], 0)
        fire_scatters(0)
        drain_stage(1)
        fire_gathers(1)

        @pl.when(i < nch2 - 1)
        def _():
            drain(ssem[0], 0)
            stage(2 * i + 2, 0)
        drain(gsem[1], 1)
        fire_scatters(1)

        @pl.when(i < nch2 - 1)
        def _():
            drain_stage(0)
            fire_gathers(0)
        return 0

    lax.fori_loop(0, nch2, pair, 0)
    drain(ssem[0], 0)
    drain(ssem[1], 1)


def _tail(src_hbm, dst_hbm, gtab, acc, dummy_hbm,
          sidx, didx, vals, gsem, ssem, tsem, row_base, nrows, nsub):
    done = 0
    while done < nrows:
        m = min(nsub, nrows - done)
        for j in range(m):
            e0 = pl.multiple_of((row_base + done + j) * SUB, SUB)
            pltpu.async_copy(src_hbm.at[pl.ds(e0, SUB)], sidx[0].at[j], tsem[0])
            pltpu.async_copy(dst_hbm.at[pl.ds(e0, SUB)], didx[0].at[j], tsem[0])
        for j in range(2 * m):
            pltpu.make_async_copy(
                src_hbm.at[pl.ds(0, SUB)], sidx[0].at[0], tsem[0]).wait()
        for j in range(m):
            pltpu.async_copy(gtab.at[sidx[0].at[j]], vals[0].at[j], gsem[0])
        for j in range(m):
            pltpu.make_async_copy(
                dummy_hbm.at[pl.ds(0, SUB)], vals[0].at[j], gsem[0]).wait()
        for j in range(m):
            pltpu.async_copy(vals[0].at[j], acc.at[didx[0].at[j]],
                             ssem[0], add=True)
        for j in range(m):
            pltpu.make_async_copy(
                dummy_hbm.at[pl.ds(0, SUB)], vals[0].at[j], ssem[0]).wait()
        done += m


def _tails(args_e, args_v, w, nsub):
    _tail(*args_e, 32 * E_MAIN + 13 * w, 13, nsub)

    @pl.when(w < 8)
    def _():
        _tail(*args_e, 32 * E_MAIN + 416 + w, 1, nsub)
    _tail(*args_v, 32 * V_MAIN + 6 * w, 6, nsub)

    @pl.when(w < 20)
    def _():
        _tail(*args_v, 32 * V_MAIN + 192 + w, 1, nsub)


def _scalar_segsum_body(x_hbm, b_hbm, eiE, eiV, comb_out,
                        sidx0, sidx1, didx0, didx1, vals0, vals1, zbuf,
                        x_sp, acc_e, acc_v, gsem0, gsem1, ssem0, ssem1,
                        tsem0, tsem1):
    cid = lax.axis_index("c")
    sid = lax.axis_index("s")

    def _zero(i, _):
        zbuf[pl.ds(pl.multiple_of(i * 16, 16), 16)] = jnp.zeros((16,), jnp.float32)
        return 0
    lax.fori_loop(0, NPT // 16, _zero, 0)
    sl = pl.ds(pl.multiple_of(sid * NPT, NPT), NPT)
    pltpu.sync_copy(zbuf, acc_e.at[sl])
    pltpu.sync_copy(zbuf, acc_v.at[sl])
    @pl.when(sid < NTILES - 1)
    def _():
        pltpu.sync_copy(x_hbm.at[sl], zbuf)

    @pl.when(sid == NTILES - 1)
    def _():
        pltpu.sync_copy(x_hbm.at[pl.ds((NTILES - 1) * NPT, NLAST)],
                        zbuf.at[pl.ds(0, NLAST)])
    pltpu.sync_copy(zbuf, x_sp.at[sl])

    @pl.when(cid == 0)
    def _():
        pltpu.sync_copy(zbuf, comb_out.at[4].at[sl])

        @pl.when(sid < NTILES - 1)
        def _():
            pltpu.sync_copy(b_hbm.at[sl], zbuf)

        @pl.when(sid == NTILES - 1)
        def _():
            pltpu.sync_copy(b_hbm.at[pl.ds((NTILES - 1) * NPT, NLAST)],
                            zbuf.at[pl.ds(0, NLAST)])
        pltpu.sync_copy(zbuf, comb_out.at[5].at[sl])
    plsc.subcore_barrier()

    w = cid * NTILES + sid
    sidx = (sidx0, sidx1)
    didx = (didx0, didx1)
    vals = (vals0, vals1)
    gsem = (gsem0, gsem1)
    ssem = (ssem0, ssem1)
    tsem = (tsem0, tsem1)
    args_e = (eiE.at[0], eiE.at[1], x_sp, acc_e, x_hbm,
              sidx, didx, vals, gsem, ssem, tsem)
    args_v = (eiV.at[0], eiV.at[1], x_sp, acc_v, x_hbm,
              sidx, didx, vals, gsem, ssem, tsem)
    _pipe(*args_e, w * E_MAIN * SUB, E_MAIN // NSUB_A, NSUB_A)
    _pipe(*args_v, w * V_MAIN * SUB, V_MAIN // NSUB_A, NSUB_A)
    _tails(args_e, args_v, w, NSUB_A)
    plsc.subcore_barrier()

    pltpu.sync_copy(acc_e.at[sl], zbuf)
    pltpu.sync_copy(zbuf, comb_out.at[cid].at[sl])
    pltpu.sync_copy(acc_v.at[sl], zbuf)
    pltpu.sync_copy(zbuf, comb_out.at[cid + 2].at[sl])


def _row_segsum_body(hd_hbm, hu_hbm, eiE, eiV, accP_out,
                     sidx0, sidx1, didx0, didx1, rows0, rows1, zrow,
                     acc, gsem0, gsem1, ssem0, ssem1, tsem0, tsem1):
    cid = lax.axis_index("c")
    sid = lax.axis_index("s")

    def _zero(i, _):
        zrow[i, :] = jnp.zeros((16,), jnp.float32)
        return 0
    lax.fori_loop(0, ZR, _zero, 0)
    for k in range(NPT // ZR):
        pltpu.sync_copy(
            zrow, acc.at[pl.ds(pl.multiple_of(sid * NPT + k * ZR, ZR), ZR)])
    plsc.subcore_barrier()

    w = cid * NTILES + sid
    sidx = (sidx0, sidx1)
    didx = (didx0, didx1)
    rows = (rows0, rows1)
    gsem = (gsem0, gsem1)
    ssem = (ssem0, ssem1)
    tsem = (tsem0, tsem1)
    args_e = (eiE.at[0], eiE.at[1], hd_hbm, acc, hd_hbm,
              sidx, didx, rows, gsem, ssem, tsem)
    args_v = (eiV.at[0], eiV.at[1], hu_hbm, acc, hu_hbm,
              sidx, didx, rows, gsem, ssem, tsem)
    _pipe(*args_e, w * E_MAIN * SUB, E_MAIN // NSUB_B, NSUB_B)
    _pipe(*args_v, w * V_MAIN * SUB, V_MAIN // NSUB_B, NSUB_B)
    _tails(args_e, args_v, w, NSUB_B)
    plsc.subcore_barrier()

    for k in range(NPT // ZR):
        slk = pl.ds(pl.multiple_of(sid * NPT + k * ZR, ZR), ZR)
        pltpu.sync_copy(acc.at[slk], zrow)
        pltpu.sync_copy(zrow, accP_out.at[cid].at[slk])


@functools.cache
def _sc_kernels():
    mesh = plsc.VectorSubcoreMesh(core_axis_name="c", subcore_axis_name="s")
    sc_params = pltpu.CompilerParams(use_tc_tiling_on_sc=False)
    scalar_segsum = pl.kernel(
        _scalar_segsum_body,
        out_type=jax.ShapeDtypeStruct((8, N_PAD), jnp.float32),
        mesh=mesh,
        scratch_types=[
            pltpu.VMEM((NSUB_A, SUB), jnp.int32),
            pltpu.VMEM((NSUB_A, SUB), jnp.int32),
            pltpu.VMEM((NSUB_A, SUB), jnp.int32),
            pltpu.VMEM((NSUB_A, SUB), jnp.int32),
            pltpu.VMEM((NSUB_A, SUB), jnp.float32),
            pltpu.VMEM((NSUB_A, SUB), jnp.float32),
            pltpu.VMEM((NPT,), jnp.float32),
            pltpu.VMEM_SHARED((N_PAD,), jnp.float32),
            pltpu.VMEM_SHARED((N_PAD,), jnp.float32),
            pltpu.VMEM_SHARED((N_PAD,), jnp.float32),
            pltpu.SemaphoreType.DMA,
            pltpu.SemaphoreType.DMA,
            pltpu.SemaphoreType.DMA,
            pltpu.SemaphoreType.DMA,
            pltpu.SemaphoreType.DMA,
            pltpu.SemaphoreType.DMA,
        ],
        compiler_params=sc_params,
    )
    row_segsum = pl.kernel(
        _row_segsum_body,
        out_type=jax.ShapeDtypeStruct((NCORES, N_PAD, H), jnp.float32),
        mesh=mesh,
        scratch_types=[
            pltpu.VMEM((NSUB_B, SUB), jnp.int32),
            pltpu.VMEM((NSUB_B, SUB), jnp.int32),
            pltpu.VMEM((NSUB_B, SUB), jnp.int32),
            pltpu.VMEM((NSUB_B, SUB), jnp.int32),
            pltpu.VMEM((NSUB_B, SUB, H), jnp.float32),
            pltpu.VMEM((NSUB_B, SUB, H), jnp.float32),
            pltpu.VMEM((ZR, H), jnp.float32),
            pltpu.VMEM_SHARED((N_PAD, H), jnp.float32),
            pltpu.SemaphoreType.DMA,
            pltpu.SemaphoreType.DMA,
            pltpu.SemaphoreType.DMA,
            pltpu.SemaphoreType.DMA,
            pltpu.SemaphoreType.DMA,
            pltpu.SemaphoreType.DMA,
        ],
        compiler_params=sc_params,
    )
    return scalar_segsum, row_segsum


NB = 16
RB = N_PAD // NB


def _h_body(comb_ref, l1t_ref, wd_ref, wu_ref, ht_ref, hd_ref, hu_ref):
    se = comb_ref[0:1, :] + comb_ref[1:2, :]
    sv = comb_ref[2:3, :] + comb_ref[3:4, :]
    xr = comb_ref[4:5, :]
    p = l1t_ref[:, 0:1]
    q = l1t_ref[:, 1:2]
    r = l1t_ref[:, 2:3]
    c = l1t_ref[:, 3:4]
    ht = jnp.maximum(p * se + q * sv + r * xr + c, 0.0)
    ht_ref[...] = ht
    hd_ref[...] = jax.lax.dot_general(ht, wd_ref[...], (((0,), (0,)), ((), ())))
    hu_ref[...] = jax.lax.dot_general(ht, wu_ref[...], (((0,), (0,)), ((), ())))


def _dense_h(comb, l1t, wd, wu):
    w16 = pl.BlockSpec((H, H), lambda i: (0, 0))
    rowspec = pl.BlockSpec((RB, H), lambda i: (i, 0))
    return pl.pallas_call(
        _h_body,
        grid=(NB,),
        in_specs=[pl.BlockSpec((8, RB), lambda i: (0, i)),
                  pl.BlockSpec((H, 8), lambda i: (0, 0)), w16, w16],
        out_specs=[pl.BlockSpec((H, RB), lambda i: (0, i)), rowspec, rowspec],
        out_shape=[jax.ShapeDtypeStruct((H, N_PAD), jnp.float32),
                   jax.ShapeDtypeStruct((N_PAD, H), jnp.float32),
                   jax.ShapeDtypeStruct((N_PAD, H), jnp.float32)],
    )(comb, l1t, wd, wu)


def _final_body(ht_ref, a0_ref, a1_ref, comb_ref, wr_ref,
                b2_ref, wc_ref, bc_ref, out_ref, acc):
    i = pl.program_id(0)

    @pl.when(i == 0)
    def _():
        acc[...] = jnp.zeros((G, H), jnp.float32)

    h2 = jnp.maximum(
        a0_ref[0] + a1_ref[0]
        + jax.lax.dot_general(ht_ref[...], wr_ref[...], (((0,), (0,)), ((), ())))
        + b2_ref[0:1, :], 0.0)
    bi = lax.bitcast_convert_type(comb_ref[5:6, :], jnp.int32)
    iot = lax.broadcasted_iota(jnp.int32, (G, 1), 0)
    rid = lax.broadcasted_iota(jnp.int32, (1, RB), 1) + i * RB
    onehot = ((bi == iot) & (rid < N)).astype(jnp.float32)
    acc[...] += jax.lax.dot_general(onehot, h2, (((1,), (0,)), ((), ())))

    @pl.when(i == NB - 1)
    def _():
        out_ref[...] = (
            jax.lax.dot_general(acc[...], wc_ref[...], (((1,), (0,)), ((), ())))
            + bc_ref[0:1, :])


def _dense_final(ht, accP, comb, wr, b2, wc_pad, bc_pad):
    return pl.pallas_call(
        _final_body,
        grid=(NB,),
        in_specs=[pl.BlockSpec((H, RB), lambda i: (0, i)),
                  pl.BlockSpec((1, RB, H), lambda i: (0, i, 0)),
                  pl.BlockSpec((1, RB, H), lambda i: (1, i, 0)),
                  pl.BlockSpec((8, RB), lambda i: (0, i)),
                  pl.BlockSpec((H, H), lambda i: (0, 0)),
                  pl.BlockSpec((8, H), lambda i: (0, 0)),
                  pl.BlockSpec((H, 128), lambda i: (0, 0)),
                  pl.BlockSpec((8, 128), lambda i: (0, 0))],
        out_specs=pl.BlockSpec((G, 128), lambda i: (0, 0)),
        out_shape=jax.ShapeDtypeStruct((G, 128), jnp.float32),
        scratch_shapes=[pltpu.VMEM((G, H), jnp.float32)],
    )(ht, accP, accP, comb, wr, b2, wc_pad, bc_pad)


def kernel(x, edge_index, v_edge_index, batch,
           W_rel_d1, b_rel_d1, W_root_d1,
           W_rel_u1, b_rel_u1, W_root_u1,
           W_rel_d2, b_rel_d2, W_root_d2,
           W_rel_u2, b_rel_u2, W_root_u2,
           W_cls, b_cls):
    xf = x[:, 0]
    batch_bits = lax.bitcast_convert_type(batch, jnp.float32)

    scalar_segsum, row_segsum = _sc_kernels()
    comb = scalar_segsum(xf, batch_bits, edge_index, v_edge_index)

    l1t = jnp.zeros((H, 8), jnp.float32)
    l1t = l1t.at[:, 0].set(W_rel_d1[0]).at[:, 1].set(W_rel_u1[0])
    l1t = l1t.at[:, 2].set(W_root_d1[0] + W_root_u1[0])
    l1t = l1t.at[:, 3].set(b_rel_d1 + b_rel_u1)

    ht, hd, hu = _dense_h(comb, l1t, W_rel_d2, W_rel_u2)

    accP = row_segsum(hd, hu, edge_index, v_edge_index)

    b2 = jnp.zeros((8, H), jnp.float32).at[0].set(b_rel_d2 + b_rel_u2)
    wc_pad = jnp.zeros((H, 128), jnp.float32).at[:, :C].set(W_cls)
    bc_pad = jnp.zeros((8, 128), jnp.float32).at[0, :C].set(b_cls)
    out = _dense_final(ht, accP, comb, W_root_d2 + W_root_u2,
                       b2, wc_pad, bc_pad)
    return out[:, :C]
